# Initial kernel scaffold; baseline (speedup 1.0000x reference)
import jax
import jax.numpy as jnp
from jax.experimental import pallas as pl


def kernel(x, edge_index, edge_weight, W1, b1, W2, b2):
    raise NotImplementedError("write your pallas kernel here")


# speedup vs baseline: 30.4733x; 30.4733x over previous
import functools

import jax
import jax.numpy as jnp
from jax import lax
from jax.experimental import pallas as pl
from jax.experimental.pallas import tpu as pltpu
from jax.experimental.pallas import tpu_sc as plsc

N = 100000
E = 1600000
D_IN = 128
HID = 16
D_OUT = 40

NC = 2
NS = 16
NW = NC * NS
CH = 1024
ROUNDS = 49
E_PAD = NW * ROUNDS * CH
CR = CH // 128
ND_PAD = 100096
NDPT = ND_PAD // NS

_mesh = plsc.VectorSubcoreMesh(core_axis_name="c", subcore_axis_name="s")


@functools.partial(
    pl.kernel,
    out_type=jax.ShapeDtypeStruct((NC * ND_PAD,), jnp.float32),
    mesh=_mesh,
    scratch_types=[
        pltpu.VMEM((CR, 128), jnp.int32),
        pltpu.VMEM((CR, 128), jnp.float32),
        pltpu.VMEM((NDPT,), jnp.float32),
        pltpu.VMEM_SHARED((ND_PAD,), jnp.float32),
        pltpu.SemaphoreType.DMA,
    ],
)
def _deg_kernel(col2d_hbm, ew2d_hbm, out_hbm, col_v, ew_v, zstage, acc, sem):
    c = lax.axis_index("c")
    s = lax.axis_index("s")
    wid = s * NC + c

    def zbody(i, carry):
        zstage[pl.ds(i * 16, 16)] = jnp.zeros((16,), jnp.float32)
        return carry

    lax.fori_loop(0, NDPT // 16, zbody, 0, unroll=8)
    base = s * NDPT
    pltpu.sync_copy(zstage, acc.at[pl.ds(base, NDPT)])
    plsc.subcore_barrier()

    def round_body(k, carry):
        ci = k * NW + wid
        pltpu.sync_copy(col2d_hbm.at[pl.ds(ci * CR, CR)], col_v)
        pltpu.sync_copy(ew2d_hbm.at[pl.ds(ci * CR, CR)], ew_v)
        for j in range(CR):
            pltpu.sync_copy(ew_v.at[j], acc.at[col_v.at[j]], add=True)
        return carry

    lax.fori_loop(0, ROUNDS, round_body, 0)
    plsc.subcore_barrier()
    pltpu.sync_copy(acc.at[pl.ds(s * NDPT, NDPT)], zstage)
    pltpu.sync_copy(zstage, out_hbm.at[pl.ds(c * ND_PAD + s * NDPT, NDPT)])


@functools.partial(
    pl.kernel,
    out_type=jax.ShapeDtypeStruct((NC, ND_PAD, HID), jnp.float32),
    mesh=_mesh,
    compiler_params=pltpu.CompilerParams(use_tc_tiling_on_sc=False),
    scratch_types=[
        pltpu.VMEM((CH,), jnp.int32),
        pltpu.VMEM((CR, 128), jnp.int32),
        pltpu.VMEM((CH,), jnp.float32),
        pltpu.VMEM((CH, HID), jnp.float32),
        pltpu.VMEM_SHARED((ND_PAD, HID), jnp.float32),
        pltpu.SemaphoreType.DMA,
    ],
)
def _edge_kernel(row_hbm, col2d_hbm, ew_hbm, g_hbm, out_hbm,
                 row_v, col_v, ew_v, msgs, acc, sem):
    c = lax.axis_index("c")
    s = lax.axis_index("s")
    wid = s * NC + c

    def zbody(i, carry):
        msgs[i, :] = jnp.zeros((HID,), jnp.float32)
        return carry

    lax.fori_loop(0, CH, zbody, 0, unroll=8)
    base = s * NDPT
    for k in range(6):
        pltpu.sync_copy(msgs, acc.at[pl.ds(base + k * CH, CH)])
    pltpu.sync_copy(msgs.at[pl.ds(0, 112)], acc.at[pl.ds(base + 6 * CH, 112)])
    plsc.subcore_barrier()

    def round_body(k, carry):
        ci = k * NW + wid
        pltpu.sync_copy(row_hbm.at[pl.ds(ci * CH, CH)], row_v)
        pltpu.sync_copy(ew_hbm.at[pl.ds(ci * CH, CH)], ew_v)
        pltpu.async_copy(g_hbm.at[row_v], msgs, sem).wait()

        def scale_body(g, carry2):
            wv = ew_v[pl.ds(g * 16, 16)]
            for j in range(16):
                e = g * 16 + j
                msgs[e, :] = msgs[e, :] * wv[j]
            return carry2

        lax.fori_loop(0, CH // 16, scale_body, 0)

        pltpu.sync_copy(col2d_hbm.at[pl.ds(ci * CR, CR)], col_v)
        for j in range(CR):
            pltpu.sync_copy(msgs.at[pl.ds(j * 128, 128)],
                            acc.at[col_v.at[j]], add=True)
        return carry

    lax.fori_loop(0, ROUNDS, round_body, 0)
    plsc.subcore_barrier()
    for k in range(6):
        pltpu.sync_copy(acc.at[pl.ds(base + k * CH, CH)], msgs)
        pltpu.sync_copy(msgs, out_hbm.at[c, pl.ds(base + k * CH, CH)])
    pltpu.sync_copy(acc.at[pl.ds(base + 6 * CH, 112)], msgs.at[pl.ds(0, 112)])
    pltpu.sync_copy(msgs.at[pl.ds(0, 112)], out_hbm.at[c, pl.ds(base + 6 * CH, 112)])


BN = 2000


def _mm1_body(x_ref, w1_ref, o_ref):
    o_ref[...] = jnp.dot(x_ref[...], w1_ref[...],
                         preferred_element_type=jnp.float32)


def _mm1(x, W1):
    return pl.pallas_call(
        _mm1_body,
        grid=(N // BN,),
        in_specs=[
            pl.BlockSpec((BN, D_IN), lambda i: (i, 0)),
            pl.BlockSpec((D_IN, HID), lambda i: (0, 0)),
        ],
        out_specs=pl.BlockSpec((BN, HID), lambda i: (i, 0)),
        out_shape=jax.ShapeDtypeStruct((N, HID), jnp.float32),
    )(x, W1)


def _scale_body(dega_ref, degb_ref, h_ref, dis_ref, g_ref):
    deg = dega_ref[...] + degb_ref[...] + 1.0
    dis = lax.rsqrt(deg)
    dis_ref[...] = dis
    g_ref[...] = h_ref[...] * dis


def _scale(dega, degb, h):
    return pl.pallas_call(
        _scale_body,
        grid=(N // BN,),
        in_specs=[
            pl.BlockSpec((BN, 1), lambda i: (i, 0)),
            pl.BlockSpec((BN, 1), lambda i: (i, 0)),
            pl.BlockSpec((BN, HID), lambda i: (i, 0)),
        ],
        out_specs=[
            pl.BlockSpec((BN, 1), lambda i: (i, 0)),
            pl.BlockSpec((BN, HID), lambda i: (i, 0)),
        ],
        out_shape=[
            jax.ShapeDtypeStruct((N, 1), jnp.float32),
            jax.ShapeDtypeStruct((N, HID), jnp.float32),
        ],
    )(dega, degb, h)


def _layer1_body(p_ref, g_ref, dis_ref, b1_ref, g2_ref):
    agg = dis_ref[...] * (p_ref[0] + p_ref[1] + g_ref[...])
    h1 = jnp.maximum(agg + b1_ref[...], 0.0)
    g2_ref[...] = dis_ref[...] * h1


def _layer1(p, g, dis, b1):
    return pl.pallas_call(
        _layer1_body,
        grid=(N // BN,),
        in_specs=[
            pl.BlockSpec((NC, BN, HID), lambda i: (0, i, 0)),
            pl.BlockSpec((BN, HID), lambda i: (i, 0)),
            pl.BlockSpec((BN, 1), lambda i: (i, 0)),
            pl.BlockSpec((1, HID), lambda i: (0, 0)),
        ],
        out_specs=pl.BlockSpec((BN, HID), lambda i: (i, 0)),
        out_shape=jax.ShapeDtypeStruct((N, HID), jnp.float32),
    )(p, g, dis, b1)


def _layer2_body(p_ref, g_ref, dis_ref, w2_ref, b2_ref, o_ref):
    agg = dis_ref[...] * (p_ref[0] + p_ref[1] + g_ref[...])
    o_ref[...] = jnp.dot(agg, w2_ref[...],
                         preferred_element_type=jnp.float32) + b2_ref[...]


def _layer2(p, g, dis, W2, b2):
    return pl.pallas_call(
        _layer2_body,
        grid=(N // BN,),
        in_specs=[
            pl.BlockSpec((NC, BN, HID), lambda i: (0, i, 0)),
            pl.BlockSpec((BN, HID), lambda i: (i, 0)),
            pl.BlockSpec((BN, 1), lambda i: (i, 0)),
            pl.BlockSpec((HID, D_OUT), lambda i: (0, 0)),
            pl.BlockSpec((1, D_OUT), lambda i: (0, 0)),
        ],
        out_specs=pl.BlockSpec((BN, D_OUT), lambda i: (i, 0)),
        out_shape=jax.ShapeDtypeStruct((N, D_OUT), jnp.float32),
    )(p, g, dis, W2, b2)


def kernel(x, edge_index, edge_weight, W1, b1, W2, b2):
    row = edge_index[0]
    col = edge_index[1]

    pad = E_PAD - E
    fill = (jnp.arange(pad, dtype=jnp.int32) * 1009) % N
    row_p = jnp.concatenate([row, fill])
    col_p = jnp.concatenate([col, fill])
    ew_p = jnp.concatenate([edge_weight, jnp.zeros((pad,), jnp.float32)])
    col2d = col_p.reshape(E_PAD // 128, 128)
    ew2d = ew_p.reshape(E_PAD // 128, 128)

    degp = _deg_kernel(col2d, ew2d).reshape(NC, ND_PAD)
    h = _mm1(x, W1)
    dis, g1 = _scale(degp[0, :N, None], degp[1, :N, None], h)
    p1 = _edge_kernel(row_p, col2d, ew_p, g1)[:, :N]
    g2 = _layer1(p1, g1, dis, b1.reshape(1, HID))
    p2 = _edge_kernel(row_p, col2d, ew_p, g2)[:, :N]
    out = _layer2(p2, g2, dis, W2, b2.reshape(1, D_OUT))
    return out


# speedup vs baseline: 44.3966x; 1.4569x over previous
import functools

import jax
import jax.numpy as jnp
from jax import lax
from jax.experimental import pallas as pl
from jax.experimental.pallas import tpu as pltpu
from jax.experimental.pallas import tpu_sc as plsc

N = 100000
E = 1600000
D_IN = 128
HID = 16
D_OUT = 40

NC = 2
NS = 16
NW = NC * NS
CH = 512
CR = CH // 128
NCH = E // CH
ROUNDS = 98
ND_PAD = 100096
NDPT = ND_PAD // NS

_mesh = plsc.VectorSubcoreMesh(core_axis_name="c", subcore_axis_name="s")


def _rsqrt16(d):
    i = plsc.bitcast(d, jnp.int32)
    y = plsc.bitcast(jnp.int32(0x5F3759DF) - (i >> 1), jnp.float32)
    for _ in range(3):
        y = y * (1.5 - 0.5 * d * y * y)
    return y


@functools.partial(
    pl.kernel,
    out_type=jax.ShapeDtypeStruct((NC * ND_PAD,), jnp.float32),
    mesh=_mesh,
    compiler_params=pltpu.CompilerParams(use_tc_tiling_on_sc=False, needs_layout_passes=False),
    scratch_types=[
        pltpu.VMEM((CR, 128), jnp.int32),
        pltpu.VMEM((CR, 128), jnp.float32),
        pltpu.VMEM((NDPT,), jnp.float32),
        pltpu.VMEM_SHARED((ND_PAD,), jnp.float32),
        pltpu.SemaphoreType.DMA,
    ],
)
def _deg_kernel(col2d_hbm, ew2d_hbm, out_hbm, col_v, ew_v, zstage, acc, sem):
    c = lax.axis_index("c")
    s = lax.axis_index("s")
    wid = s * NC + c

    def zbody(i, carry):
        zstage[pl.ds(i * 16, 16)] = jnp.zeros((16,), jnp.float32)
        return carry

    lax.fori_loop(0, NDPT // 16, zbody, 0, unroll=8)
    base = s * NDPT
    pltpu.sync_copy(zstage, acc.at[pl.ds(base, NDPT)])
    plsc.subcore_barrier()

    def round_body(k, carry):
        ci = k * NW + wid

        @pl.when(ci < NCH)
        def _():
            pltpu.sync_copy(col2d_hbm.at[pl.ds(ci * CR, CR)], col_v)
            pltpu.sync_copy(ew2d_hbm.at[pl.ds(ci * CR, CR)], ew_v)
            for j in range(CR):
                pltpu.sync_copy(ew_v.at[j], acc.at[col_v.at[j]], add=True)

        return carry

    lax.fori_loop(0, ROUNDS, round_body, 0)
    plsc.subcore_barrier()
    pltpu.sync_copy(acc.at[pl.ds(base, NDPT)], zstage)
    pltpu.sync_copy(zstage, out_hbm.at[pl.ds(c * ND_PAD + base, NDPT)])


def _make_edge_kernel(first):
    if first:
        out_type = [
            jax.ShapeDtypeStruct((NC, ND_PAD, HID), jnp.float32),
            jax.ShapeDtypeStruct((ND_PAD, HID), jnp.float32),
            jax.ShapeDtypeStruct((ND_PAD,), jnp.float32),
        ]
    else:
        out_type = [
            jax.ShapeDtypeStruct((NC, ND_PAD, HID), jnp.float32),
            jax.ShapeDtypeStruct((ND_PAD, HID), jnp.float32),
        ]

    scratch = [
        pltpu.VMEM((CH,), jnp.int32), pltpu.VMEM((CH,), jnp.int32),
        pltpu.VMEM((CR, 128), jnp.int32), pltpu.VMEM((CR, 128), jnp.int32),
        pltpu.VMEM((CH,), jnp.float32), pltpu.VMEM((CH,), jnp.float32),
        pltpu.VMEM((CH, HID), jnp.float32), pltpu.VMEM((CH, HID), jnp.float32),
        pltpu.VMEM((CH,), jnp.float32),
        pltpu.VMEM((CH,), jnp.float32),
        pltpu.VMEM((16,), jnp.float32),
        pltpu.VMEM_SHARED((ND_PAD, HID), jnp.float32),
        pltpu.SemaphoreType.DMA, pltpu.SemaphoreType.DMA,
        pltpu.SemaphoreType.DMA, pltpu.SemaphoreType.DMA,
        pltpu.SemaphoreType.DMA, pltpu.SemaphoreType.DMA,
    ]

    def body(*refs):
        if first:
            (row_hbm, col2d_hbm, ew_hbm, degp_hbm, h_hbm,
             p_hbm, g_hbm, dis_hbm,
             rv0, rv1, cv0, cv1, wv0, wv1, m0, m1, dbuf, disbuf, b1buf,
             acc, is0, is1, gs0, gs1, ss0, ss1) = refs
        else:
            (row_hbm, col2d_hbm, ew_hbm, pprev_hbm, gprev_hbm, disp_hbm,
             b1_hbm,
             p_hbm, g_hbm,
             rv0, rv1, cv0, cv1, wv0, wv1, m0, m1, dbuf, disbuf, b1buf,
             acc, is0, is1, gs0, gs1, ss0, ss1) = refs

        row_v = [rv0, rv1]
        col_v = [cv0, cv1]
        ew_v = [wv0, wv1]
        msgs = [m0, m1]
        isem = [is0, is1]
        gsem = [gs0, gs1]
        ssem = [ss0, ss1]

        c = lax.axis_index("c")
        s = lax.axis_index("s")
        wid = s * NC + c
        base = s * NDPT

        if first:
            def pchunk(off, sz):
                pltpu.sync_copy(h_hbm.at[pl.ds(off, sz)],
                                msgs[0].at[pl.ds(0, sz)])
                pltpu.sync_copy(degp_hbm.at[pl.ds(off, sz)],
                                ew_v[0].at[pl.ds(0, sz)])
                pltpu.sync_copy(degp_hbm.at[pl.ds(ND_PAD + off, sz)],
                                dbuf.at[pl.ds(0, sz)])

                def gbody(g, carry):
                    dv = (ew_v[0][pl.ds(g * 16, 16)]
                          + dbuf[pl.ds(g * 16, 16)] + 1.0)
                    y = _rsqrt16(dv)
                    disbuf[pl.ds(g * 16, 16)] = y
                    for j in range(16):
                        e = g * 16 + j
                        msgs[0][e, :] = msgs[0][e, :] * y[j]
                    return carry

                lax.fori_loop(0, sz // 16, gbody, 0)
                pltpu.sync_copy(disbuf.at[pl.ds(0, sz)],
                                dis_hbm.at[pl.ds(off, sz)])
                pltpu.sync_copy(msgs[0].at[pl.ds(0, sz)],
                                g_hbm.at[pl.ds(off, sz)])

            def prol_body(k, carry):
                pchunk(base + k * CH, CH)
                return carry

            lax.fori_loop(0, 12, prol_body, 0)
            pchunk(base + 12 * CH, 112)
        else:
            pltpu.sync_copy(b1_hbm, b1buf)
            b1v = b1buf[...]

            def pchunk(off, sz):
                pltpu.sync_copy(pprev_hbm.at[0, pl.ds(off, sz)],
                                msgs[0].at[pl.ds(0, sz)])
                pltpu.sync_copy(pprev_hbm.at[1, pl.ds(off, sz)],
                                msgs[1].at[pl.ds(0, sz)])

                def addbody(r, carry):
                    msgs[0][r, :] = msgs[0][r, :] + msgs[1][r, :]
                    return carry

                lax.fori_loop(0, sz, addbody, 0, unroll=8)
                pltpu.sync_copy(gprev_hbm.at[pl.ds(off, sz)],
                                msgs[1].at[pl.ds(0, sz)])
                pltpu.sync_copy(disp_hbm.at[pl.ds(off, sz)],
                                disbuf.at[pl.ds(0, sz)])

                def gbody(g, carry):
                    y = disbuf[pl.ds(g * 16, 16)]
                    for j in range(16):
                        e = g * 16 + j
                        v = (msgs[0][e, :] + msgs[1][e, :]) * y[j] + b1v
                        v = jnp.maximum(v, 0.0) * y[j]
                        msgs[0][e, :] = v
                    return carry

                lax.fori_loop(0, sz // 16, gbody, 0)
                pltpu.sync_copy(msgs[0].at[pl.ds(0, sz)],
                                g_hbm.at[pl.ds(off, sz)])

            def prol_body(k, carry):
                pchunk(base + k * CH, CH)
                return carry

            lax.fori_loop(0, 12, prol_body, 0)
            pchunk(base + 12 * CH, 112)

        def zbody(i, carry):
            msgs[0][i, :] = jnp.zeros((HID,), jnp.float32)
            return carry

        lax.fori_loop(0, CH, zbody, 0, unroll=8)
        for k in range(12):
            pltpu.sync_copy(msgs[0], acc.at[pl.ds(base + k * CH, CH)])
        pltpu.sync_copy(msgs[0].at[pl.ds(0, 112)],
                        acc.at[pl.ds(base + 12 * CH, 112)])
        plsc.subcore_barrier()

        def start_in(rr, b):
            ci = jnp.minimum(rr * NW + wid, NCH - 1)
            pltpu.async_copy(row_hbm.at[pl.ds(ci * CH, CH)], row_v[b], isem[b])
            pltpu.async_copy(ew_hbm.at[pl.ds(ci * CH, CH)], ew_v[b], isem[b])
            pltpu.async_copy(col2d_hbm.at[pl.ds(ci * CR, CR)], col_v[b],
                             isem[b])

        def wait_in(b):
            pltpu.make_async_copy(row_hbm.at[pl.ds(0, CH)], row_v[b],
                                  isem[b]).wait()
            pltpu.make_async_copy(ew_hbm.at[pl.ds(0, CH)], ew_v[b],
                                  isem[b]).wait()
            pltpu.make_async_copy(col2d_hbm.at[pl.ds(0, CR)], col_v[b],
                                  isem[b]).wait()

        start_in(0, 0)
        start_in(1, 1)
        wait_in(0)
        pltpu.async_copy(g_hbm.at[row_v[0]], msgs[0], gsem[0])

        def pair_body(r2, carry):
            for sub in (0, 1):
                b = sub
                nb = 1 - b
                r = r2 * 2 + sub
                pltpu.make_async_copy(g_hbm.at[pl.ds(0, CH)], msgs[b],
                                      gsem[b]).wait()
                ci = r * NW + wid
                factor = jnp.where(ci < NCH, 1.0, 0.0)

                def scale_body(g, carry2):
                    wv = ew_v[b][pl.ds(g * 16, 16)] * factor
                    for j in range(16):
                        e = g * 16 + j
                        msgs[b][e, :] = msgs[b][e, :] * wv[j]
                    return carry2

                lax.fori_loop(0, CH // 16, scale_body, 0)
                for j in range(CR):
                    pltpu.async_copy(msgs[b].at[pl.ds(j * 128, 128)],
                                     acc.at[col_v[b].at[j]], ssem[b],
                                     add=True)
                wait_in(nb)
                pltpu.async_copy(g_hbm.at[row_v[nb]], msgs[nb], gsem[nb])
                pltpu.make_async_copy(g_hbm.at[pl.ds(0, CH)], msgs[b],
                                      ssem[b]).wait()
                start_in(r + 2, b)
            return carry

        lax.fori_loop(0, ROUNDS // 2, pair_body, 0)
        pltpu.make_async_copy(g_hbm.at[pl.ds(0, CH)], msgs[0], gsem[0]).wait()
        wait_in(1)
        plsc.subcore_barrier()

        for k in range(12):
            pltpu.sync_copy(acc.at[pl.ds(base + k * CH, CH)], msgs[0])
            pltpu.sync_copy(msgs[0], p_hbm.at[c, pl.ds(base + k * CH, CH)])
        pltpu.sync_copy(acc.at[pl.ds(base + 12 * CH, 112)],
                        msgs[0].at[pl.ds(0, 112)])
        pltpu.sync_copy(msgs[0].at[pl.ds(0, 112)],
                        p_hbm.at[c, pl.ds(base + 12 * CH, 112)])

    return pl.kernel(
        body,
        out_type=out_type,
        mesh=_mesh,
        compiler_params=pltpu.CompilerParams(use_tc_tiling_on_sc=False, needs_layout_passes=False),
        scratch_types=scratch,
    )


_edge1 = _make_edge_kernel(True)
_edge2 = _make_edge_kernel(False)


BN = 2000


def _mm1_body(x_ref, w1_ref, o_ref):
    o_ref[...] = jnp.dot(x_ref[...], w1_ref[...],
                         preferred_element_type=jnp.float32)


def _mm1(x, W1):
    return pl.pallas_call(
        _mm1_body,
        grid=(N // BN,),
        in_specs=[
            pl.BlockSpec((BN, D_IN), lambda i: (i, 0)),
            pl.BlockSpec((D_IN, HID), lambda i: (0, 0)),
        ],
        out_specs=pl.BlockSpec((BN, HID), lambda i: (i, 0)),
        out_shape=jax.ShapeDtypeStruct((ND_PAD, HID), jnp.float32),
    )(x, W1)


def _layer2_body(p_ref, g_ref, dis_ref, w2_ref, b2_ref, o_ref):
    agg = dis_ref[...] * (p_ref[0] + p_ref[1] + g_ref[...])
    o_ref[...] = jnp.dot(agg, w2_ref[...],
                         preferred_element_type=jnp.float32) + b2_ref[...]


def _layer2(p, g, dis, W2, b2):
    return pl.pallas_call(
        _layer2_body,
        grid=(N // BN,),
        in_specs=[
            pl.BlockSpec((NC, BN, HID), lambda i: (0, i, 0)),
            pl.BlockSpec((BN, HID), lambda i: (i, 0)),
            pl.BlockSpec((BN, 1), lambda i: (i, 0)),
            pl.BlockSpec((HID, D_OUT), lambda i: (0, 0)),
            pl.BlockSpec((1, D_OUT), lambda i: (0, 0)),
        ],
        out_specs=pl.BlockSpec((BN, D_OUT), lambda i: (i, 0)),
        out_shape=jax.ShapeDtypeStruct((N, D_OUT), jnp.float32),
    )(p, g, dis, W2, b2)


def kernel(x, edge_index, edge_weight, W1, b1, W2, b2):
    row = edge_index[0]
    col = edge_index[1]
    col2d = col.reshape(E // 128, 128)
    ew2d = edge_weight.reshape(E // 128, 128)

    degp = _deg_kernel(col2d, ew2d)
    h = _mm1(x, W1)
    p1, g1, dis = _edge1(row, col2d, edge_weight, degp, h)
    p2, g2 = _edge2(row, col2d, edge_weight, p1, g1, dis, b1)
    out = _layer2(p2, g2, dis.reshape(ND_PAD, 1), W2,
                  b2.reshape(1, D_OUT))
    return out


# speedup vs baseline: 46.6679x; 1.0512x over previous
import functools

import jax
import jax.numpy as jnp
from jax import lax
from jax.experimental import pallas as pl
from jax.experimental.pallas import tpu as pltpu
from jax.experimental.pallas import tpu_sc as plsc

N = 100000
E = 1600000
D_IN = 128
HID = 16
D_OUT = 40

NC = 2
NS = 16
NW = NC * NS
CH = 512
CR = CH // 128
NCH = E // CH
ROUNDS = 102
CHD = 2560
CRD = CHD // 128
NCHD = E // CHD
ROUNDS_D = 20
ND_PAD = 100096
NDPT = ND_PAD // NS

_mesh = plsc.VectorSubcoreMesh(core_axis_name="c", subcore_axis_name="s")
_sc_params = pltpu.CompilerParams(use_tc_tiling_on_sc=False,
                                  needs_layout_passes=False)


def _rsqrt16(d):
    i = plsc.bitcast(d, jnp.int32)
    y = plsc.bitcast(jnp.int32(0x5F3759DF) - (i >> 1), jnp.float32)
    for _ in range(3):
        y = y * (1.5 - 0.5 * d * y * y)
    return y


@functools.partial(
    pl.kernel,
    out_type=jax.ShapeDtypeStruct((NC * ND_PAD,), jnp.float32),
    mesh=_mesh,
    compiler_params=_sc_params,
    scratch_types=[
        pltpu.VMEM((CRD, 128), jnp.int32),
        pltpu.VMEM((CRD, 128), jnp.float32),
        pltpu.VMEM((NDPT,), jnp.float32),
        pltpu.VMEM_SHARED((ND_PAD,), jnp.float32),
        pltpu.SemaphoreType.DMA,
    ],
)
def _deg_kernel(ei3_hbm, ew2d_hbm, out_hbm, col_v, ew_v, zstage, acc, sem):
    c = lax.axis_index("c")
    s = lax.axis_index("s")
    wid = s * NC + c

    def zbody(i, carry):
        zstage[pl.ds(i * 16, 16)] = jnp.zeros((16,), jnp.float32)
        return carry

    lax.fori_loop(0, NDPT // 16, zbody, 0, unroll=8)
    base = s * NDPT
    pltpu.sync_copy(zstage, acc.at[pl.ds(base, NDPT)])
    plsc.subcore_barrier()

    def round_body(k, carry):
        ci = k * NW + wid

        @pl.when(ci < NCHD)
        def _():
            pltpu.sync_copy(ei3_hbm.at[1, pl.ds(ci * CRD, CRD)], col_v)
            pltpu.sync_copy(ew2d_hbm.at[pl.ds(ci * CRD, CRD)], ew_v)
            for j in range(CRD):
                pltpu.async_copy(ew_v.at[j], acc.at[col_v.at[j]], sem,
                                 add=True)
            pltpu.make_async_copy(ew2d_hbm.at[pl.ds(0, CRD)], ew_v,
                                  sem).wait()

        return carry

    lax.fori_loop(0, ROUNDS_D, round_body, 0)
    plsc.subcore_barrier()
    pltpu.sync_copy(acc.at[pl.ds(base, NDPT)], zstage)
    pltpu.sync_copy(zstage, out_hbm.at[pl.ds(c * ND_PAD + base, NDPT)])


def _make_edge_kernel(first):
    if first:
        out_type = [
            jax.ShapeDtypeStruct((NC, ND_PAD, HID), jnp.float32),
            jax.ShapeDtypeStruct((ND_PAD, HID), jnp.float32),
            jax.ShapeDtypeStruct((ND_PAD,), jnp.float32),
        ]
    else:
        out_type = [
            jax.ShapeDtypeStruct((NC, ND_PAD, HID), jnp.float32),
            jax.ShapeDtypeStruct((ND_PAD, HID), jnp.float32),
        ]

    scratch = [
        pltpu.VMEM((CH,), jnp.int32), pltpu.VMEM((CH,), jnp.int32),
        pltpu.VMEM((CR, 128), jnp.int32), pltpu.VMEM((CR, 128), jnp.int32),
        pltpu.VMEM((CR, 128), jnp.int32),
        pltpu.VMEM((CH,), jnp.float32), pltpu.VMEM((CH,), jnp.float32),
        pltpu.VMEM((CH, HID), jnp.float32), pltpu.VMEM((CH, HID), jnp.float32),
        pltpu.VMEM((CH, HID), jnp.float32),
        pltpu.VMEM((CH,), jnp.float32),
        pltpu.VMEM((CH,), jnp.float32),
        pltpu.VMEM((16,), jnp.float32),
        pltpu.VMEM_SHARED((ND_PAD, HID), jnp.float32),
        pltpu.SemaphoreType.DMA, pltpu.SemaphoreType.DMA,
        pltpu.SemaphoreType.DMA, pltpu.SemaphoreType.DMA,
        pltpu.SemaphoreType.DMA, pltpu.SemaphoreType.DMA,
    ]

    def body(*refs):
        if first:
            (row_hbm, ei3_hbm, ew_hbm, degp_hbm, h_hbm,
             p_hbm, g_hbm, dis_hbm,
             rv0, rv1, cv0, cv1, cv2, wv0, wv1, m0, m1, gbuf, dbuf, disbuf,
             b1buf, acc, is0, is1, gs0, gs1, ss0, ss1) = refs
        else:
            (row_hbm, ei3_hbm, ew_hbm, pprev_hbm, gprev_hbm, disp_hbm,
             b1_hbm,
             p_hbm, g_hbm,
             rv0, rv1, cv0, cv1, cv2, wv0, wv1, m0, m1, gbuf, dbuf, disbuf,
             b1buf, acc, is0, is1, gs0, gs1, ss0, ss1) = refs

        row_v = [rv0, rv1]
        col_s = [cv0, cv1, cv2]
        ew_v = [wv0, wv1]
        msgs = [m0, m1]
        isem = [is0, is1]
        gsem = [gs0, gs1]
        ssem = [ss0, ss1]

        c = lax.axis_index("c")
        s = lax.axis_index("s")
        wid = s * NC + c
        base = s * NDPT

        if first:
            def pchunk(off, sz):
                pltpu.sync_copy(h_hbm.at[pl.ds(off, sz)],
                                msgs[0].at[pl.ds(0, sz)])
                pltpu.sync_copy(degp_hbm.at[pl.ds(off, sz)],
                                ew_v[0].at[pl.ds(0, sz)])
                pltpu.sync_copy(degp_hbm.at[pl.ds(ND_PAD + off, sz)],
                                dbuf.at[pl.ds(0, sz)])

                def gbody(g, carry):
                    dv = (ew_v[0][pl.ds(g * 16, 16)]
                          + dbuf[pl.ds(g * 16, 16)] + 1.0)
                    y = _rsqrt16(dv)
                    disbuf[pl.ds(g * 16, 16)] = y
                    for j in range(16):
                        e = g * 16 + j
                        msgs[0][e, :] = msgs[0][e, :] * y[j]
                    return carry

                lax.fori_loop(0, sz // 16, gbody, 0)
                pltpu.sync_copy(disbuf.at[pl.ds(0, sz)],
                                dis_hbm.at[pl.ds(off, sz)])
                pltpu.sync_copy(msgs[0].at[pl.ds(0, sz)],
                                g_hbm.at[pl.ds(off, sz)])

            def prol_body(k, carry):
                pchunk(base + k * CH, CH)
                return carry

            lax.fori_loop(0, 12, prol_body, 0)
            pchunk(base + 12 * CH, 112)
        else:
            pltpu.sync_copy(b1_hbm, b1buf)
            b1v = b1buf[...]

            def pchunk(off, sz):
                pltpu.sync_copy(pprev_hbm.at[0, pl.ds(off, sz)],
                                msgs[0].at[pl.ds(0, sz)])
                pltpu.sync_copy(pprev_hbm.at[1, pl.ds(off, sz)],
                                msgs[1].at[pl.ds(0, sz)])
                pltpu.sync_copy(gprev_hbm.at[pl.ds(off, sz)],
                                gbuf.at[pl.ds(0, sz)])
                pltpu.sync_copy(disp_hbm.at[pl.ds(off, sz)],
                                disbuf.at[pl.ds(0, sz)])

                def gbody(g, carry):
                    y = disbuf[pl.ds(g * 16, 16)]
                    for j in range(16):
                        e = g * 16 + j
                        v = (msgs[0][e, :] + msgs[1][e, :]
                             + gbuf[e, :]) * y[j] + b1v
                        v = jnp.maximum(v, 0.0) * y[j]
                        msgs[0][e, :] = v
                    return carry

                lax.fori_loop(0, sz // 16, gbody, 0)
                pltpu.sync_copy(msgs[0].at[pl.ds(0, sz)],
                                g_hbm.at[pl.ds(off, sz)])

            def prol_body(k, carry):
                pchunk(base + k * CH, CH)
                return carry

            lax.fori_loop(0, 12, prol_body, 0)
            pchunk(base + 12 * CH, 112)

        def zbody(i, carry):
            msgs[0][i, :] = jnp.zeros((HID,), jnp.float32)
            return carry

        lax.fori_loop(0, CH, zbody, 0, unroll=8)
        for k in range(12):
            pltpu.sync_copy(msgs[0], acc.at[pl.ds(base + k * CH, CH)])
        pltpu.sync_copy(msgs[0].at[pl.ds(0, 112)],
                        acc.at[pl.ds(base + 12 * CH, 112)])
        plsc.subcore_barrier()

        def start_in(rr, b, cs):
            ci = jnp.minimum(rr * NW + wid, NCH - 1)
            pltpu.async_copy(row_hbm.at[pl.ds(ci * CH, CH)], row_v[b], isem[b])
            pltpu.async_copy(ew_hbm.at[pl.ds(ci * CH, CH)], ew_v[b], isem[b])
            pltpu.async_copy(ei3_hbm.at[1, pl.ds(ci * CR, CR)], col_s[cs],
                             isem[b])

        def wait_in(b, cs):
            pltpu.make_async_copy(row_hbm.at[pl.ds(0, CH)], row_v[b],
                                  isem[b]).wait()
            pltpu.make_async_copy(ew_hbm.at[pl.ds(0, CH)], ew_v[b],
                                  isem[b]).wait()
            pltpu.make_async_copy(ei3_hbm.at[1, pl.ds(0, CR)], col_s[cs],
                                  isem[b]).wait()

        start_in(0, 0, 0)
        start_in(1, 1, 1)
        wait_in(0, 0)
        pltpu.async_copy(g_hbm.at[row_v[0]], msgs[0], gsem[0])

        def six_body(r6, carry):
            for sub in range(6):
                b = sub % 2
                nb = 1 - b
                cs = sub % 3
                r = r6 * 6 + sub
                pltpu.make_async_copy(g_hbm.at[pl.ds(0, CH)], msgs[b],
                                      gsem[b]).wait()
                ci = r * NW + wid
                factor = jnp.where(ci < NCH, 1.0, 0.0)

                def scale_body(g, carry2):
                    wv = ew_v[b][pl.ds(g * 16, 16)] * factor
                    for j in range(16):
                        e = g * 16 + j
                        msgs[b][e, :] = msgs[b][e, :] * wv[j]
                    return carry2

                lax.fori_loop(0, CH // 16, scale_body, 0)
                for j in range(CR):
                    pltpu.async_copy(msgs[b].at[pl.ds(j * 128, 128)],
                                     acc.at[col_s[cs].at[j]], ssem[b],
                                     add=True)
                wait_in(nb, (cs + 1) % 3)
                if sub == 0:
                    @pl.when(r6 > 0)
                    def _():
                        pltpu.make_async_copy(g_hbm.at[pl.ds(0, CH)],
                                              msgs[nb], ssem[nb]).wait()
                else:
                    pltpu.make_async_copy(g_hbm.at[pl.ds(0, CH)], msgs[nb],
                                          ssem[nb]).wait()
                pltpu.async_copy(g_hbm.at[row_v[nb]], msgs[nb], gsem[nb])
                start_in(r + 2, b, (cs + 2) % 3)
            return carry

        lax.fori_loop(0, ROUNDS // 6, six_body, 0)
        pltpu.make_async_copy(g_hbm.at[pl.ds(0, CH)], msgs[1], ssem[1]).wait()
        pltpu.make_async_copy(g_hbm.at[pl.ds(0, CH)], msgs[0], gsem[0]).wait()
        wait_in(1, 1)
        plsc.subcore_barrier()

        for k in range(12):
            pltpu.sync_copy(acc.at[pl.ds(base + k * CH, CH)], msgs[0])
            pltpu.sync_copy(msgs[0], p_hbm.at[c, pl.ds(base + k * CH, CH)])
        pltpu.sync_copy(acc.at[pl.ds(base + 12 * CH, 112)],
                        msgs[0].at[pl.ds(0, 112)])
        pltpu.sync_copy(msgs[0].at[pl.ds(0, 112)],
                        p_hbm.at[c, pl.ds(base + 12 * CH, 112)])

    return pl.kernel(
        body,
        out_type=out_type,
        mesh=_mesh,
        compiler_params=_sc_params,
        scratch_types=scratch,
    )


_edge1 = _make_edge_kernel(True)
_edge2 = _make_edge_kernel(False)


BN = 2000


def _mm1_body(x_ref, w1_ref, o_ref):
    o_ref[...] = jnp.dot(x_ref[...], w1_ref[...],
                         preferred_element_type=jnp.float32)


def _mm1(x, W1):
    return pl.pallas_call(
        _mm1_body,
        grid=(N // BN,),
        in_specs=[
            pl.BlockSpec((BN, D_IN), lambda i: (i, 0)),
            pl.BlockSpec((D_IN, HID), lambda i: (0, 0)),
        ],
        out_specs=pl.BlockSpec((BN, HID), lambda i: (i, 0)),
        out_shape=jax.ShapeDtypeStruct((ND_PAD, HID), jnp.float32),
    )(x, W1)


def _layer2_body(p_ref, g_ref, dis_ref, w2_ref, b2_ref, o_ref):
    agg = dis_ref[...] * (p_ref[0] + p_ref[1] + g_ref[...])
    o_ref[...] = jnp.dot(agg, w2_ref[...],
                         preferred_element_type=jnp.float32) + b2_ref[...]


def _layer2(p, g, dis, W2, b2):
    return pl.pallas_call(
        _layer2_body,
        grid=(N // BN,),
        in_specs=[
            pl.BlockSpec((NC, BN, HID), lambda i: (0, i, 0)),
            pl.BlockSpec((BN, HID), lambda i: (i, 0)),
            pl.BlockSpec((BN, 1), lambda i: (i, 0)),
            pl.BlockSpec((HID, D_OUT), lambda i: (0, 0)),
            pl.BlockSpec((1, D_OUT), lambda i: (0, 0)),
        ],
        out_specs=pl.BlockSpec((BN, D_OUT), lambda i: (i, 0)),
        out_shape=jax.ShapeDtypeStruct((N, D_OUT), jnp.float32),
    )(p, g, dis, W2, b2)


def kernel(x, edge_index, edge_weight, W1, b1, W2, b2):
    row = edge_index[0]
    ei3 = edge_index.reshape(2, E // 128, 128)
    ew2d = edge_weight.reshape(E // 128, 128)

    degp = _deg_kernel(ei3, ew2d)
    h = _mm1(x, W1)
    p1, g1, dis = _edge1(row, ei3, edge_weight, degp, h)
    p2, g2 = _edge2(row, ei3, edge_weight, p1, g1, dis, b1)
    out = _layer2(p2, g2, dis.reshape(ND_PAD, 1), W2,
                  b2.reshape(1, D_OUT))
    return out


# speedup vs baseline: 55.9089x; 1.1980x over previous
import functools

import jax
import jax.numpy as jnp
from jax import lax
from jax.experimental import pallas as pl
from jax.experimental.pallas import tpu as pltpu
from jax.experimental.pallas import tpu_sc as plsc

N = 100000
E = 1600000
D_IN = 128
HID = 16
D_OUT = 40

NC = 2
NS = 16
NW = NC * NS
CH = 512
CR = CH // 128
NCH = E // CH
ROUNDS = 102
CHD = 2560
CRD = CHD // 128
NCHD = E // CHD
ROUNDS_D = 20
ND_PAD = 100096
NDPT = ND_PAD // NS
PC = 256
PNF = NDPT // PC
PREM = NDPT - PNF * PC

_mesh = plsc.VectorSubcoreMesh(core_axis_name="c", subcore_axis_name="s")
_sc_params = pltpu.CompilerParams(use_tc_tiling_on_sc=False,
                                  needs_layout_passes=False)


def _rsqrt16(d):
    i = plsc.bitcast(d, jnp.int32)
    y = plsc.bitcast(jnp.int32(0x5F3759DF) - (i >> 1), jnp.float32)
    for _ in range(3):
        y = y * (1.5 - 0.5 * d * y * y)
    return y


@functools.partial(
    pl.kernel,
    out_type=jax.ShapeDtypeStruct((NC * ND_PAD,), jnp.float32),
    mesh=_mesh,
    compiler_params=_sc_params,
    scratch_types=[
        pltpu.VMEM((CRD, 128), jnp.int32),
        pltpu.VMEM((CRD, 128), jnp.float32),
        pltpu.VMEM((NDPT,), jnp.float32),
        pltpu.VMEM_SHARED((ND_PAD,), jnp.float32),
        pltpu.SemaphoreType.DMA,
    ],
)
def _deg_kernel(ei3_hbm, ew2d_hbm, out_hbm, col_v, ew_v, zstage, acc, sem):
    c = lax.axis_index("c")
    s = lax.axis_index("s")
    wid = s * NC + c

    def zbody(i, carry):
        zstage[pl.ds(i * 16, 16)] = jnp.zeros((16,), jnp.float32)
        return carry

    lax.fori_loop(0, NDPT // 16, zbody, 0, unroll=8)
    base = s * NDPT
    pltpu.sync_copy(zstage, acc.at[pl.ds(base, NDPT)])
    plsc.subcore_barrier()

    def round_body(k, carry):
        ci = k * NW + wid

        @pl.when(ci < NCHD)
        def _():
            pltpu.sync_copy(ei3_hbm.at[1, pl.ds(ci * CRD, CRD)], col_v)
            pltpu.sync_copy(ew2d_hbm.at[pl.ds(ci * CRD, CRD)], ew_v)
            for j in range(CRD):
                pltpu.async_copy(ew_v.at[j], acc.at[col_v.at[j]], sem,
                                 add=True)
            pltpu.make_async_copy(ew2d_hbm.at[pl.ds(0, CRD)], ew_v,
                                  sem).wait()

        return carry

    lax.fori_loop(0, ROUNDS_D, round_body, 0)
    plsc.subcore_barrier()
    pltpu.sync_copy(acc.at[pl.ds(base, NDPT)], zstage)
    pltpu.sync_copy(zstage, out_hbm.at[pl.ds(c * ND_PAD + base, NDPT)])


def _make_edge_kernel(first):
    out_type = [
        jax.ShapeDtypeStruct((ND_PAD, HID), jnp.float32),
        jax.ShapeDtypeStruct((ND_PAD, HID), jnp.float32),
        jax.ShapeDtypeStruct((ND_PAD, HID), jnp.float32),
        jax.ShapeDtypeStruct((ND_PAD, HID), jnp.float32),
    ]
    if not first:
        out_type = out_type[:3]

    scratch = [
        pltpu.VMEM((CH,), jnp.int32), pltpu.VMEM((CH,), jnp.int32),
        pltpu.VMEM((CR, 128), jnp.int32), pltpu.VMEM((CR, 128), jnp.int32),
        pltpu.VMEM((CR, 128), jnp.int32),
        pltpu.VMEM((CH,), jnp.float32), pltpu.VMEM((CH,), jnp.float32),
        pltpu.VMEM((CH, HID), jnp.float32), pltpu.VMEM((CH, HID), jnp.float32),
        pltpu.VMEM((CH, HID), jnp.float32),
        pltpu.VMEM((CH,), jnp.float32),
        pltpu.VMEM((CH,), jnp.float32),
        pltpu.VMEM((16,), jnp.float32),
        pltpu.VMEM_SHARED((ND_PAD, HID), jnp.float32),
        pltpu.SemaphoreType.DMA, pltpu.SemaphoreType.DMA,
        pltpu.SemaphoreType.DMA, pltpu.SemaphoreType.DMA,
        pltpu.SemaphoreType.DMA, pltpu.SemaphoreType.DMA,
    ]

    def body(*refs):
        if first:
            (row_hbm, ei3_hbm, ew_hbm, degp_hbm, h_hbm,
             p0_hbm, p1_hbm, g_hbm, de_hbm,
             rv0, rv1, cv0, cv1, cv2, wv0, wv1, m0, m1, gbuf, dbuf, disbuf,
             b1buf, acc, is0, is1, gs0, gs1, ss0, ss1) = refs
        else:
            (row_hbm, ei3_hbm, ew_hbm, pp0_hbm, pp1_hbm, de_hbm, b1_hbm,
             p0_hbm, p1_hbm, g_hbm,
             rv0, rv1, cv0, cv1, cv2, wv0, wv1, m0, m1, gbuf, dbuf, disbuf,
             b1buf, acc, is0, is1, gs0, gs1, ss0, ss1) = refs

        row_v = [rv0, rv1]
        col_s = [cv0, cv1, cv2]
        ew_v = [wv0, wv1]
        msgs = [m0, m1]
        isem = [is0, is1]
        gsem = [gs0, gs1]
        ssem = [ss0, ss1]

        c = lax.axis_index("c")
        s = lax.axis_index("s")
        wid = s * NC + c
        base = s * NDPT

        nchunks = PNF + 1

        def psz(k):
            return PC if k < PNF else PREM

        if first:
            def start_pin(k, b):
                off = base + k * PC
                sz = psz(k)
                pltpu.async_copy(h_hbm.at[pl.ds(off, sz)],
                                 msgs[b].at[pl.ds(0, sz)], isem[b])
                pltpu.async_copy(degp_hbm.at[pl.ds(off, sz)],
                                 ew_v[b].at[pl.ds(0, sz)], isem[b])
                pltpu.async_copy(degp_hbm.at[pl.ds(ND_PAD + off, sz)],
                                 (dbuf if b == 0 else disbuf).at[pl.ds(0, sz)],
                                 isem[b])

            def wait_pin(k, b):
                sz = psz(k)
                pltpu.make_async_copy(h_hbm.at[pl.ds(0, sz)],
                                      msgs[b].at[pl.ds(0, sz)],
                                      isem[b]).wait()
                pltpu.make_async_copy(degp_hbm.at[pl.ds(0, sz)],
                                      ew_v[b].at[pl.ds(0, sz)],
                                      isem[b]).wait()
                pltpu.make_async_copy(degp_hbm.at[pl.ds(0, sz)],
                                      (dbuf if b == 0 else
                                       disbuf).at[pl.ds(0, sz)],
                                      isem[b]).wait()

            def compute_p(k, b):
                sz = psz(k)
                db = dbuf if b == 0 else disbuf

                def gbody(g, carry):
                    dv = (ew_v[b][pl.ds(g * 16, 16)]
                          + db[pl.ds(g * 16, 16)] + 1.0)
                    y = _rsqrt16(dv)
                    for j in range(16):
                        e = g * 16 + j
                        msgs[b][PC + e, :] = jnp.zeros((HID,),
                                                       jnp.float32) + y[j]
                        msgs[b][e, :] = msgs[b][e, :] * y[j]
                    return carry

                lax.fori_loop(0, sz // 16, gbody, 0)

            def fire_pout(k, b):
                off = base + k * PC
                sz = psz(k)
                pltpu.async_copy(msgs[b].at[pl.ds(0, sz)],
                                 g_hbm.at[pl.ds(off, sz)], gsem[b])
                pltpu.async_copy(msgs[b].at[pl.ds(PC, sz)],
                                 de_hbm.at[pl.ds(off, sz)], gsem[b])

                @pl.when(c == 0)
                def _():
                    pltpu.sync_copy(msgs[b].at[pl.ds(0, sz)],
                                    acc.at[pl.ds(off, sz)])

            def drain_pout(k, b):
                sz = psz(k)
                pltpu.make_async_copy(g_hbm.at[pl.ds(0, sz)],
                                      msgs[b].at[pl.ds(0, sz)],
                                      gsem[b]).wait()
                pltpu.make_async_copy(g_hbm.at[pl.ds(0, sz)],
                                      msgs[b].at[pl.ds(PC, sz)],
                                      gsem[b]).wait()
        else:
            pltpu.sync_copy(b1_hbm, b1buf)

            def start_pin(k, b):
                off = base + k * PC
                sz = psz(k)
                pltpu.async_copy(pp0_hbm.at[pl.ds(off, sz)],
                                 msgs[b].at[pl.ds(0, sz)], isem[b])
                pltpu.async_copy(pp1_hbm.at[pl.ds(off, sz)],
                                 gbuf.at[pl.ds(b * PC, sz)], isem[b])
                pltpu.async_copy(de_hbm.at[pl.ds(off, sz)],
                                 msgs[b].at[pl.ds(PC, sz)], isem[b])

            def wait_pin(k, b):
                sz = psz(k)
                pltpu.make_async_copy(g_hbm.at[pl.ds(0, sz)],
                                      msgs[b].at[pl.ds(0, sz)],
                                      isem[b]).wait()
                pltpu.make_async_copy(g_hbm.at[pl.ds(0, sz)],
                                      gbuf.at[pl.ds(b * PC, sz)],
                                      isem[b]).wait()
                pltpu.make_async_copy(g_hbm.at[pl.ds(0, sz)],
                                      msgs[b].at[pl.ds(PC, sz)],
                                      isem[b]).wait()

            def compute_p(k, b):
                sz = psz(k)
                b1v = b1buf[...]

                def gbody(g, carry):
                    for j in range(16):
                        e = g * 16 + j
                        y = msgs[b][PC + e, :]
                        v = (msgs[b][e, :] + gbuf[b * PC + e, :]) * y + b1v
                        v = jnp.maximum(v, 0.0) * y
                        msgs[b][e, :] = v
                    return carry

                lax.fori_loop(0, sz // 16, gbody, 0)

            def fire_pout(k, b):
                off = base + k * PC
                sz = psz(k)
                pltpu.async_copy(msgs[b].at[pl.ds(0, sz)],
                                 g_hbm.at[pl.ds(off, sz)], gsem[b])

                @pl.when(c == 0)
                def _():
                    pltpu.sync_copy(msgs[b].at[pl.ds(0, sz)],
                                    acc.at[pl.ds(off, sz)])

            def drain_pout(k, b):
                sz = psz(k)
                pltpu.make_async_copy(g_hbm.at[pl.ds(0, sz)],
                                      msgs[b].at[pl.ds(0, sz)],
                                      gsem[b]).wait()

        start_pin(0, 0)
        for k in range(nchunks):
            b = k % 2
            wait_pin(k, b)
            if k >= 1:
                drain_pout(k - 1, 1 - b)
            if k + 1 < nchunks:
                start_pin(k + 1, 1 - b)
            compute_p(k, b)
            fire_pout(k, b)
        drain_pout(nchunks - 1, (nchunks - 1) % 2)

        @pl.when(c == 1)
        def _():
            def zbody(i, carry):
                msgs[0][i, :] = jnp.zeros((HID,), jnp.float32)
                return carry

            lax.fori_loop(0, CH, zbody, 0, unroll=8)
            for k in range(12):
                pltpu.sync_copy(msgs[0], acc.at[pl.ds(base + k * CH, CH)])
            pltpu.sync_copy(msgs[0].at[pl.ds(0, 112)],
                            acc.at[pl.ds(base + 12 * CH, 112)])

        plsc.subcore_barrier()

        def start_in(rr, b, cs):
            ci = jnp.minimum(rr * NW + wid, NCH - 1)
            pltpu.async_copy(row_hbm.at[pl.ds(ci * CH, CH)], row_v[b], isem[b])
            pltpu.async_copy(ew_hbm.at[pl.ds(ci * CH, CH)], ew_v[b], isem[b])
            pltpu.async_copy(ei3_hbm.at[1, pl.ds(ci * CR, CR)], col_s[cs],
                             isem[b])

        def wait_in(b, cs):
            pltpu.make_async_copy(row_hbm.at[pl.ds(0, CH)], row_v[b],
                                  isem[b]).wait()
            pltpu.make_async_copy(ew_hbm.at[pl.ds(0, CH)], ew_v[b],
                                  isem[b]).wait()
            pltpu.make_async_copy(ei3_hbm.at[1, pl.ds(0, CR)], col_s[cs],
                                  isem[b]).wait()

        start_in(0, 0, 0)
        start_in(1, 1, 1)
        wait_in(0, 0)
        pltpu.async_copy(g_hbm.at[row_v[0]], msgs[0], gsem[0])

        def six_body(r6, carry):
            for sub in range(6):
                b = sub % 2
                nb = 1 - b
                cs = sub % 3
                r = r6 * 6 + sub
                pltpu.make_async_copy(g_hbm.at[pl.ds(0, CH)], msgs[b],
                                      gsem[b]).wait()
                ci = r * NW + wid
                factor = jnp.where(ci < NCH, 1.0, 0.0)

                def scale_body(g, carry2):
                    wv = ew_v[b][pl.ds(g * 16, 16)] * factor
                    for j in range(16):
                        e = g * 16 + j
                        msgs[b][e, :] = msgs[b][e, :] * wv[j]
                    return carry2

                lax.fori_loop(0, CH // 16, scale_body, 0)
                for j in range(CR):
                    pltpu.async_copy(msgs[b].at[pl.ds(j * 128, 128)],
                                     acc.at[col_s[cs].at[j]], ssem[b],
                                     add=True)
                wait_in(nb, (cs + 1) % 3)
                if sub == 0:
                    @pl.when(r6 > 0)
                    def _():
                        pltpu.make_async_copy(g_hbm.at[pl.ds(0, CH)],
                                              msgs[nb], ssem[nb]).wait()
                else:
                    pltpu.make_async_copy(g_hbm.at[pl.ds(0, CH)], msgs[nb],
                                          ssem[nb]).wait()
                pltpu.async_copy(g_hbm.at[row_v[nb]], msgs[nb], gsem[nb])
                start_in(r + 2, b, (cs + 2) % 3)
            return carry

        lax.fori_loop(0, ROUNDS // 6, six_body, 0)
        pltpu.make_async_copy(g_hbm.at[pl.ds(0, CH)], msgs[1], ssem[1]).wait()
        pltpu.make_async_copy(g_hbm.at[pl.ds(0, CH)], msgs[0], gsem[0]).wait()
        wait_in(1, 1)
        plsc.subcore_barrier()

        myp = [p0_hbm, p1_hbm]
        for cc in range(NC):
            @pl.when(c == cc)
            def _(cc=cc):
                for k in range(12):
                    pltpu.sync_copy(acc.at[pl.ds(base + k * CH, CH)], msgs[0])
                    pltpu.sync_copy(msgs[0],
                                    myp[cc].at[pl.ds(base + k * CH, CH)])
                pltpu.sync_copy(acc.at[pl.ds(base + 12 * CH, 112)],
                                msgs[0].at[pl.ds(0, 112)])
                pltpu.sync_copy(msgs[0].at[pl.ds(0, 112)],
                                myp[cc].at[pl.ds(base + 12 * CH, 112)])

    return pl.kernel(
        body,
        out_type=out_type,
        mesh=_mesh,
        compiler_params=_sc_params,
        scratch_types=scratch,
    )


_edge1 = _make_edge_kernel(True)
_edge2 = _make_edge_kernel(False)


BN = 2000


def _mm1_body(x_ref, w1_ref, o_ref):
    o_ref[...] = jnp.dot(x_ref[...], w1_ref[...],
                         preferred_element_type=jnp.float32)


def _mm1(x, W1):
    return pl.pallas_call(
        _mm1_body,
        grid=(N // BN,),
        in_specs=[
            pl.BlockSpec((BN, D_IN), lambda i: (i, 0)),
            pl.BlockSpec((D_IN, HID), lambda i: (0, 0)),
        ],
        out_specs=pl.BlockSpec((BN, HID), lambda i: (i, 0)),
        out_shape=jax.ShapeDtypeStruct((ND_PAD, HID), jnp.float32),
    )(x, W1)


FB = 250
FR = N * HID // 128
FRP = ND_PAD * HID // 128


def _fin_body(p0_ref, p1_ref, de_ref, w2_ref, b2_ref, o_ref):
    agg = (p0_ref[...] + p1_ref[...]) * de_ref[...]
    o_ref[...] = jnp.dot(agg, w2_ref[...],
                         preferred_element_type=jnp.float32) + b2_ref[...]


def _final(p0, p1, de, W2big, b2big):
    return pl.pallas_call(
        _fin_body,
        grid=(1,),
        in_specs=[
            pl.BlockSpec((FRP, 128), lambda i: (0, 0)),
            pl.BlockSpec((FRP, 128), lambda i: (0, 0)),
            pl.BlockSpec((FRP, 128), lambda i: (0, 0)),
            pl.BlockSpec((128, 8 * D_OUT), lambda i: (0, 0)),
            pl.BlockSpec((1, 8 * D_OUT), lambda i: (0, 0)),
        ],
        out_specs=pl.BlockSpec((FRP, 8 * D_OUT), lambda i: (0, 0)),
        out_shape=jax.ShapeDtypeStruct((FRP, 8 * D_OUT), jnp.float32),
    )(p0, p1, de, W2big, b2big)


def kernel(x, edge_index, edge_weight, W1, b1, W2, b2):
    row = edge_index[0]
    ei3 = edge_index.reshape(2, E // 128, 128)
    ew2d = edge_weight.reshape(E // 128, 128)

    degp = _deg_kernel(ei3, ew2d)
    h = _mm1(x, W1)
    p10, p11, g1, de = _edge1(row, ei3, edge_weight, degp, h)
    p20, p21, g2 = _edge2(row, ei3, edge_weight, p10, p11, de, b1)
    W2big = jnp.kron(jnp.eye(8, dtype=jnp.float32), W2)
    b2big = jnp.tile(b2, 8).reshape(1, 8 * D_OUT)
    out = _final(p20.reshape(FRP, 128), p21.reshape(FRP, 128),
                 de.reshape(FRP, 128), W2big, b2big)
    return out.reshape(ND_PAD, D_OUT)[:N]


# speedup vs baseline: 61.6105x; 1.1020x over previous
import functools

import jax
import jax.numpy as jnp
from jax import lax
from jax.experimental import pallas as pl
from jax.experimental.pallas import tpu as pltpu
from jax.experimental.pallas import tpu_sc as plsc

N = 100000
E = 1600000
D_IN = 128
HID = 16
D_OUT = 40

NC = 2
NS = 16
NW = NC * NS
CH = 512
CR = CH // 128
NCH = E // CH
ROUNDS = 102
CHD = 2560
CRD = CHD // 128
NCHD = E // CHD
ROUNDS_D = 20
ND_PAD = 100096
NDPT = ND_PAD // NS
PC = 256
PNF = NDPT // PC
PREM = NDPT - PNF * PC

_mesh = plsc.VectorSubcoreMesh(core_axis_name="c", subcore_axis_name="s")
_sc_params = pltpu.CompilerParams(use_tc_tiling_on_sc=False,
                                  needs_layout_passes=False)


def _rsqrt16(d):
    i = plsc.bitcast(d, jnp.int32)
    y = plsc.bitcast(jnp.int32(0x5F3759DF) - (i >> 1), jnp.float32)
    for _ in range(3):
        y = y * (1.5 - 0.5 * d * y * y)
    return y


@functools.partial(
    pl.kernel,
    out_type=jax.ShapeDtypeStruct((NC * ND_PAD,), jnp.float32),
    mesh=_mesh,
    compiler_params=_sc_params,
    scratch_types=[
        pltpu.VMEM((CRD, 128), jnp.int32),
        pltpu.VMEM((CRD, 128), jnp.float32),
        pltpu.VMEM((NDPT,), jnp.float32),
        pltpu.VMEM_SHARED((ND_PAD,), jnp.float32),
        pltpu.SemaphoreType.DMA,
    ],
)
def _deg_kernel(ei3_hbm, ew2d_hbm, out_hbm, col_v, ew_v, zstage, acc, sem):
    c = lax.axis_index("c")
    s = lax.axis_index("s")
    wid = s * NC + c

    def zbody(i, carry):
        zstage[pl.ds(i * 16, 16)] = jnp.zeros((16,), jnp.float32)
        return carry

    lax.fori_loop(0, NDPT // 16, zbody, 0, unroll=8)
    base = s * NDPT
    pltpu.sync_copy(zstage, acc.at[pl.ds(base, NDPT)])
    plsc.subcore_barrier()

    def round_body(k, carry):
        ci = k * NW + wid

        @pl.when(ci < NCHD)
        def _():
            pltpu.sync_copy(ei3_hbm.at[1, pl.ds(ci * CRD, CRD)], col_v)
            pltpu.sync_copy(ew2d_hbm.at[pl.ds(ci * CRD, CRD)], ew_v)
            for j in range(CRD):
                pltpu.async_copy(ew_v.at[j], acc.at[col_v.at[j]], sem,
                                 add=True)
            pltpu.make_async_copy(ew2d_hbm.at[pl.ds(0, CRD)], ew_v,
                                  sem).wait()

        return carry

    lax.fori_loop(0, ROUNDS_D, round_body, 0)
    plsc.subcore_barrier()
    pltpu.sync_copy(acc.at[pl.ds(base, NDPT)], zstage)
    pltpu.sync_copy(zstage, out_hbm.at[pl.ds(c * ND_PAD + base, NDPT)])


def _make_edge_kernel(first):
    out_type = [
        jax.ShapeDtypeStruct((ND_PAD, HID), jnp.float32),
        jax.ShapeDtypeStruct((ND_PAD, HID), jnp.float32),
        jax.ShapeDtypeStruct((ND_PAD, HID), jnp.float32),
        jax.ShapeDtypeStruct((ND_PAD, HID), jnp.float32),
    ]
    if not first:
        out_type = out_type[:3]

    scratch = [
        pltpu.VMEM((CR, 128), jnp.int32), pltpu.VMEM((CR, 128), jnp.int32),
        pltpu.VMEM((CR, 128), jnp.int32), pltpu.VMEM((CR, 128), jnp.int32),
        pltpu.VMEM((CR, 128), jnp.int32),
        pltpu.VMEM((CR, 128), jnp.float32), pltpu.VMEM((CR, 128), jnp.float32),
        pltpu.VMEM((CH, HID), jnp.float32), pltpu.VMEM((CH, HID), jnp.float32),
        pltpu.VMEM((CH, HID), jnp.float32),
        pltpu.VMEM((CH,), jnp.float32),
        pltpu.VMEM((CH,), jnp.float32),
        pltpu.VMEM((16,), jnp.float32),
        pltpu.VMEM_SHARED((ND_PAD, HID), jnp.float32),
        pltpu.SemaphoreType.DMA, pltpu.SemaphoreType.DMA,
        pltpu.SemaphoreType.DMA, pltpu.SemaphoreType.DMA,
        pltpu.SemaphoreType.DMA, pltpu.SemaphoreType.DMA,
    ]

    def body(*refs):
        if first:
            (ei3_hbm, ew2d_hbm, degp_hbm, h_hbm,
             p0_hbm, p1_hbm, g_hbm, de_hbm,
             rv0, rv1, cv0, cv1, cv2, wv0, wv1, m0, m1, gbuf, dbuf, disbuf,
             b1buf, acc, is0, is1, gs0, gs1, ss0, ss1) = refs
        else:
            (ei3_hbm, ew2d_hbm, pp0_hbm, pp1_hbm, de_hbm, b1_hbm,
             p0_hbm, p1_hbm, g_hbm,
             rv0, rv1, cv0, cv1, cv2, wv0, wv1, m0, m1, gbuf, dbuf, disbuf,
             b1buf, acc, is0, is1, gs0, gs1, ss0, ss1) = refs

        row_v = [rv0, rv1]
        col_s = [cv0, cv1, cv2]
        ew_v = [wv0, wv1]
        msgs = [m0, m1]
        isem = [is0, is1]
        gsem = [gs0, gs1]
        ssem = [ss0, ss1]

        c = lax.axis_index("c")
        s = lax.axis_index("s")
        wid = s * NC + c
        base = s * NDPT

        nchunks = PNF + 1

        def psz(k):
            return PC if k < PNF else PREM

        if first:
            def start_pin(k, b):
                off = base + k * PC
                sz = psz(k)
                pltpu.async_copy(h_hbm.at[pl.ds(off, sz)],
                                 msgs[b].at[pl.ds(0, sz)], isem[b])
                pltpu.async_copy(degp_hbm.at[pl.ds(off, sz)],
                                 dbuf.at[pl.ds(b * PC, sz)], isem[b])
                pltpu.async_copy(degp_hbm.at[pl.ds(ND_PAD + off, sz)],
                                 disbuf.at[pl.ds(b * PC, sz)], isem[b])

            def wait_pin(k, b):
                sz = psz(k)
                pltpu.make_async_copy(h_hbm.at[pl.ds(0, sz)],
                                      msgs[b].at[pl.ds(0, sz)],
                                      isem[b]).wait()
                pltpu.make_async_copy(degp_hbm.at[pl.ds(0, sz)],
                                      dbuf.at[pl.ds(b * PC, sz)],
                                      isem[b]).wait()
                pltpu.make_async_copy(degp_hbm.at[pl.ds(0, sz)],
                                      disbuf.at[pl.ds(b * PC, sz)],
                                      isem[b]).wait()

            def compute_p(k, b):
                sz = psz(k)

                def gbody(g, carry):
                    dv = (dbuf[pl.ds(b * PC + g * 16, 16)]
                          + disbuf[pl.ds(b * PC + g * 16, 16)] + 1.0)
                    y = _rsqrt16(dv)
                    for j in range(16):
                        e = g * 16 + j
                        msgs[b][PC + e, :] = jnp.zeros((HID,),
                                                       jnp.float32) + y[j]
                        msgs[b][e, :] = msgs[b][e, :] * y[j]
                    return carry

                lax.fori_loop(0, sz // 16, gbody, 0)

            def fire_pout(k, b):
                off = base + k * PC
                sz = psz(k)
                pltpu.async_copy(msgs[b].at[pl.ds(0, sz)],
                                 g_hbm.at[pl.ds(off, sz)], gsem[b])
                pltpu.async_copy(msgs[b].at[pl.ds(PC, sz)],
                                 de_hbm.at[pl.ds(off, sz)], gsem[b])

                @pl.when(c == 0)
                def _():
                    pltpu.sync_copy(msgs[b].at[pl.ds(0, sz)],
                                    acc.at[pl.ds(off, sz)])

            def drain_pout(k, b):
                sz = psz(k)
                pltpu.make_async_copy(g_hbm.at[pl.ds(0, sz)],
                                      msgs[b].at[pl.ds(0, sz)],
                                      gsem[b]).wait()
                pltpu.make_async_copy(g_hbm.at[pl.ds(0, sz)],
                                      msgs[b].at[pl.ds(PC, sz)],
                                      gsem[b]).wait()
        else:
            pltpu.sync_copy(b1_hbm, b1buf)

            def start_pin(k, b):
                off = base + k * PC
                sz = psz(k)
                pltpu.async_copy(pp0_hbm.at[pl.ds(off, sz)],
                                 msgs[b].at[pl.ds(0, sz)], isem[b])
                pltpu.async_copy(pp1_hbm.at[pl.ds(off, sz)],
                                 gbuf.at[pl.ds(b * PC, sz)], isem[b])
                pltpu.async_copy(de_hbm.at[pl.ds(off, sz)],
                                 msgs[b].at[pl.ds(PC, sz)], isem[b])

            def wait_pin(k, b):
                sz = psz(k)
                pltpu.make_async_copy(g_hbm.at[pl.ds(0, sz)],
                                      msgs[b].at[pl.ds(0, sz)],
                                      isem[b]).wait()
                pltpu.make_async_copy(g_hbm.at[pl.ds(0, sz)],
                                      gbuf.at[pl.ds(b * PC, sz)],
                                      isem[b]).wait()
                pltpu.make_async_copy(g_hbm.at[pl.ds(0, sz)],
                                      msgs[b].at[pl.ds(PC, sz)],
                                      isem[b]).wait()

            def compute_p(k, b):
                sz = psz(k)
                b1v = b1buf[...]

                def gbody(g, carry):
                    for j in range(16):
                        e = g * 16 + j
                        y = msgs[b][PC + e, :]
                        v = (msgs[b][e, :] + gbuf[b * PC + e, :]) * y + b1v
                        v = jnp.maximum(v, 0.0) * y
                        msgs[b][e, :] = v
                    return carry

                lax.fori_loop(0, sz // 16, gbody, 0)

            def fire_pout(k, b):
                off = base + k * PC
                sz = psz(k)
                pltpu.async_copy(msgs[b].at[pl.ds(0, sz)],
                                 g_hbm.at[pl.ds(off, sz)], gsem[b])

                @pl.when(c == 0)
                def _():
                    pltpu.sync_copy(msgs[b].at[pl.ds(0, sz)],
                                    acc.at[pl.ds(off, sz)])

            def drain_pout(k, b):
                sz = psz(k)
                pltpu.make_async_copy(g_hbm.at[pl.ds(0, sz)],
                                      msgs[b].at[pl.ds(0, sz)],
                                      gsem[b]).wait()

        start_pin(0, 0)
        for k in range(nchunks):
            b = k % 2
            wait_pin(k, b)
            if k >= 1:
                drain_pout(k - 1, 1 - b)
            if k + 1 < nchunks:
                start_pin(k + 1, 1 - b)
            compute_p(k, b)
            fire_pout(k, b)
        drain_pout(nchunks - 1, (nchunks - 1) % 2)

        @pl.when(c == 1)
        def _():
            def zbody(i, carry):
                msgs[0][i, :] = jnp.zeros((HID,), jnp.float32)
                return carry

            lax.fori_loop(0, CH, zbody, 0, unroll=8)
            for k in range(12):
                pltpu.sync_copy(msgs[0], acc.at[pl.ds(base + k * CH, CH)])
            pltpu.sync_copy(msgs[0].at[pl.ds(0, 112)],
                            acc.at[pl.ds(base + 12 * CH, 112)])

        plsc.subcore_barrier()

        def start_in(rr, b, cs):
            ci = jnp.minimum(rr * NW + wid, NCH - 1)
            pltpu.async_copy(ei3_hbm.at[0, pl.ds(ci * CR, CR)], row_v[b],
                             isem[b])
            pltpu.async_copy(ew2d_hbm.at[pl.ds(ci * CR, CR)], ew_v[b],
                             isem[b])
            pltpu.async_copy(ei3_hbm.at[1, pl.ds(ci * CR, CR)], col_s[cs],
                             isem[b])

        def wait_in(b, cs):
            pltpu.make_async_copy(ei3_hbm.at[0, pl.ds(0, CR)], row_v[b],
                                  isem[b]).wait()
            pltpu.make_async_copy(ew2d_hbm.at[pl.ds(0, CR)], ew_v[b],
                                  isem[b]).wait()
            pltpu.make_async_copy(ei3_hbm.at[1, pl.ds(0, CR)], col_s[cs],
                                  isem[b]).wait()

        def start_gather(b):
            for j in range(CR):
                pltpu.async_copy(g_hbm.at[row_v[b].at[j]],
                                 msgs[b].at[pl.ds(j * 128, 128)], gsem[b])

        start_in(0, 0, 0)
        start_in(1, 1, 1)
        wait_in(0, 0)
        start_gather(0)

        def six_body(r6, carry):
            for sub in range(6):
                b = sub % 2
                nb = 1 - b
                cs = sub % 3
                r = r6 * 6 + sub
                pltpu.make_async_copy(g_hbm.at[pl.ds(0, CH)], msgs[b],
                                      gsem[b]).wait()
                ci = r * NW + wid
                factor = jnp.where(ci < NCH, 1.0, 0.0)

                def scale_body(g, carry2):
                    jr = g // 8
                    g2 = g % 8
                    wv = ew_v[b][jr, pl.ds(g2 * 16, 16)] * factor
                    for j in range(16):
                        e = g * 16 + j
                        msgs[b][e, :] = msgs[b][e, :] * wv[j]
                    return carry2

                lax.fori_loop(0, CH // 16, scale_body, 0)
                for j in range(CR):
                    pltpu.async_copy(msgs[b].at[pl.ds(j * 128, 128)],
                                     acc.at[col_s[cs].at[j]], ssem[b],
                                     add=True)
                wait_in(nb, (cs + 1) % 3)
                if sub == 0:
                    @pl.when(r6 > 0)
                    def _():
                        pltpu.make_async_copy(g_hbm.at[pl.ds(0, CH)],
                                              msgs[nb], ssem[nb]).wait()
                else:
                    pltpu.make_async_copy(g_hbm.at[pl.ds(0, CH)], msgs[nb],
                                          ssem[nb]).wait()
                start_gather(nb)
                start_in(r + 2, b, (cs + 2) % 3)
            return carry

        lax.fori_loop(0, ROUNDS // 6, six_body, 0)
        pltpu.make_async_copy(g_hbm.at[pl.ds(0, CH)], msgs[1], ssem[1]).wait()
        pltpu.make_async_copy(g_hbm.at[pl.ds(0, CH)], msgs[0], gsem[0]).wait()
        wait_in(1, 1)
        plsc.subcore_barrier()

        myp = [p0_hbm, p1_hbm]
        for cc in range(NC):
            @pl.when(c == cc)
            def _(cc=cc):
                for k in range(12):
                    pltpu.sync_copy(acc.at[pl.ds(base + k * CH, CH)], msgs[0])
                    pltpu.sync_copy(msgs[0],
                                    myp[cc].at[pl.ds(base + k * CH, CH)])
                pltpu.sync_copy(acc.at[pl.ds(base + 12 * CH, 112)],
                                msgs[0].at[pl.ds(0, 112)])
                pltpu.sync_copy(msgs[0].at[pl.ds(0, 112)],
                                myp[cc].at[pl.ds(base + 12 * CH, 112)])

    return pl.kernel(
        body,
        out_type=out_type,
        mesh=_mesh,
        compiler_params=_sc_params,
        scratch_types=scratch,
    )


_edge1 = _make_edge_kernel(True)
_edge2 = _make_edge_kernel(False)


BN = 2000


def _mm1_body(x_ref, w1_ref, o_ref):
    o_ref[...] = jnp.dot(x_ref[...], w1_ref[...],
                         preferred_element_type=jnp.float32)


def _mm1(x, W1):
    return pl.pallas_call(
        _mm1_body,
        grid=(N // BN,),
        in_specs=[
            pl.BlockSpec((BN, D_IN), lambda i: (i, 0)),
            pl.BlockSpec((D_IN, HID), lambda i: (0, 0)),
        ],
        out_specs=pl.BlockSpec((BN, HID), lambda i: (i, 0)),
        out_shape=jax.ShapeDtypeStruct((ND_PAD, HID), jnp.float32),
    )(x, W1)


FB = 250
FR = N * HID // 128
FRP = ND_PAD * HID // 128


def _fin_body(p0_ref, p1_ref, de_ref, w2_ref, b2_ref, o_ref):
    agg = (p0_ref[...] + p1_ref[...]) * de_ref[...]
    o_ref[...] = jnp.dot(agg, w2_ref[...],
                         preferred_element_type=jnp.float32) + b2_ref[...]


def _final(p0, p1, de, W2big, b2big):
    return pl.pallas_call(
        _fin_body,
        grid=(1,),
        in_specs=[
            pl.BlockSpec((FRP, 128), lambda i: (0, 0)),
            pl.BlockSpec((FRP, 128), lambda i: (0, 0)),
            pl.BlockSpec((FRP, 128), lambda i: (0, 0)),
            pl.BlockSpec((128, 8 * D_OUT), lambda i: (0, 0)),
            pl.BlockSpec((1, 8 * D_OUT), lambda i: (0, 0)),
        ],
        out_specs=pl.BlockSpec((FRP, 8 * D_OUT), lambda i: (0, 0)),
        out_shape=jax.ShapeDtypeStruct((FRP, 8 * D_OUT), jnp.float32),
    )(p0, p1, de, W2big, b2big)


def kernel(x, edge_index, edge_weight, W1, b1, W2, b2):
    ei3 = edge_index.reshape(2, E // 128, 128)
    ew2d = edge_weight.reshape(E // 128, 128)

    degp = _deg_kernel(ei3, ew2d)
    h = _mm1(x, W1)
    p10, p11, g1, de = _edge1(ei3, ew2d, degp, h)
    p20, p21, g2 = _edge2(ei3, ew2d, p10, p11, de, b1)
    W2big = jnp.kron(jnp.eye(8, dtype=jnp.float32), W2)
    b2big = jnp.tile(b2, 8).reshape(1, 8 * D_OUT)
    out = _final(p20.reshape(FRP, 128), p21.reshape(FRP, 128),
                 de.reshape(FRP, 128), W2big, b2big)
    return out.reshape(ND_PAD, D_OUT)[:N]


# speedup vs baseline: 62.3723x; 1.0124x over previous
import functools

import jax
import jax.numpy as jnp
from jax import lax
from jax.experimental import pallas as pl
from jax.experimental.pallas import tpu as pltpu
from jax.experimental.pallas import tpu_sc as plsc

N = 100000
E = 1600000
D_IN = 128
HID = 16
D_OUT = 40

NC = 2
NS = 16
NW = NC * NS
CH = 512
CR = CH // 128
NCH = E // CH
ROUNDS = 102
CHD = 2560
CRD = CHD // 128
NCHD = E // CHD
ROUNDS_D = 20
ND_PAD = 100096
NDPT = ND_PAD // NS
PC = 256
PNF = NDPT // PC
PREM = NDPT - PNF * PC

_mesh = plsc.VectorSubcoreMesh(core_axis_name="c", subcore_axis_name="s")
_sc_params = pltpu.CompilerParams(use_tc_tiling_on_sc=False,
                                  needs_layout_passes=False)


def _rsqrt16(d):
    i = plsc.bitcast(d, jnp.int32)
    y = plsc.bitcast(jnp.int32(0x5F3759DF) - (i >> 1), jnp.float32)
    for _ in range(3):
        y = y * (1.5 - 0.5 * d * y * y)
    return y


@functools.partial(
    pl.kernel,
    out_type=jax.ShapeDtypeStruct((NC * ND_PAD,), jnp.float32),
    mesh=_mesh,
    compiler_params=_sc_params,
    scratch_types=[
        pltpu.VMEM((CRD, 128), jnp.int32),
        pltpu.VMEM((CRD, 128), jnp.float32),
        pltpu.VMEM((NDPT,), jnp.float32),
        pltpu.VMEM_SHARED((ND_PAD,), jnp.float32),
        pltpu.SemaphoreType.DMA,
    ],
)
def _deg_kernel(ei3_hbm, ew2d_hbm, out_hbm, col_v, ew_v, zstage, acc, sem):
    c = lax.axis_index("c")
    s = lax.axis_index("s")
    wid = s * NC + c

    def zbody(i, carry):
        zstage[pl.ds(i * 16, 16)] = jnp.zeros((16,), jnp.float32)
        return carry

    lax.fori_loop(0, NDPT // 16, zbody, 0, unroll=8)
    base = s * NDPT
    pltpu.sync_copy(zstage, acc.at[pl.ds(base, NDPT)])
    plsc.subcore_barrier()

    def round_body(k, carry):
        ci = k * NW + wid

        @pl.when(ci < NCHD)
        def _():
            pltpu.sync_copy(ei3_hbm.at[1, pl.ds(ci * CRD, CRD)], col_v)
            pltpu.sync_copy(ew2d_hbm.at[pl.ds(ci * CRD, CRD)], ew_v)
            for j in range(CRD):
                pltpu.async_copy(ew_v.at[j], acc.at[col_v.at[j]], sem,
                                 add=True)
            pltpu.make_async_copy(ew2d_hbm.at[pl.ds(0, CRD)], ew_v,
                                  sem).wait()

        return carry

    lax.fori_loop(0, ROUNDS_D, round_body, 0)
    plsc.subcore_barrier()
    pltpu.sync_copy(acc.at[pl.ds(base, NDPT)], zstage)
    pltpu.sync_copy(zstage, out_hbm.at[pl.ds(c * ND_PAD + base, NDPT)])


def _make_edge_kernel(first):
    out_type = [
        jax.ShapeDtypeStruct((ND_PAD, HID), jnp.float32),
        jax.ShapeDtypeStruct((ND_PAD, HID), jnp.float32),
        jax.ShapeDtypeStruct((ND_PAD, HID), jnp.float32),
        jax.ShapeDtypeStruct((ND_PAD, HID), jnp.float32),
    ]
    if not first:
        out_type = out_type[:3]

    scratch = [
        pltpu.VMEM((CR, 128), jnp.int32), pltpu.VMEM((CR, 128), jnp.int32),
        pltpu.VMEM((CR, 128), jnp.int32), pltpu.VMEM((CR, 128), jnp.int32),
        pltpu.VMEM((CR, 128), jnp.int32),
        pltpu.VMEM((CR, 128), jnp.float32), pltpu.VMEM((CR, 128), jnp.float32),
        pltpu.VMEM((CH, HID), jnp.float32), pltpu.VMEM((CH, HID), jnp.float32),
        pltpu.VMEM((CH, HID), jnp.float32),
        pltpu.VMEM((CH,), jnp.float32),
        pltpu.VMEM((CH,), jnp.float32),
        pltpu.VMEM((16,), jnp.float32),
        pltpu.VMEM_SHARED((ND_PAD, HID), jnp.float32),
        pltpu.SemaphoreType.DMA, pltpu.SemaphoreType.DMA,
        pltpu.SemaphoreType.DMA, pltpu.SemaphoreType.DMA,
        pltpu.SemaphoreType.DMA, pltpu.SemaphoreType.DMA,
    ]

    def body(*refs):
        if first:
            (ei3_hbm, ew2d_hbm, degp_hbm, h_hbm,
             p0_hbm, p1_hbm, g_hbm, de_hbm,
             rv0, rv1, cv0, cv1, cv2, wv0, wv1, m0, m1, gbuf, dbuf, disbuf,
             b1buf, acc, is0, is1, gs0, gs1, ss0, ss1) = refs
        else:
            (ei3_hbm, ew2d_hbm, pp0_hbm, pp1_hbm, de_hbm, b1_hbm,
             p0_hbm, p1_hbm, g_hbm,
             rv0, rv1, cv0, cv1, cv2, wv0, wv1, m0, m1, gbuf, dbuf, disbuf,
             b1buf, acc, is0, is1, gs0, gs1, ss0, ss1) = refs

        row_v = [rv0, rv1]
        col_s = [cv0, cv1, cv2]
        ew_v = [wv0, wv1]
        msgs = [m0, m1]
        isem = [is0, is1]
        gsem = [gs0, gs1]
        ssem = [ss0, ss1]

        c = lax.axis_index("c")
        s = lax.axis_index("s")
        wid = s * NC + c
        base = s * NDPT

        nchunks = PNF + 1

        def psz(k):
            return PC if k < PNF else PREM

        if first:
            def start_pin(k, b):
                off = base + k * PC
                sz = psz(k)
                pltpu.async_copy(h_hbm.at[pl.ds(off, sz)],
                                 msgs[b].at[pl.ds(0, sz)], isem[b])
                pltpu.async_copy(degp_hbm.at[pl.ds(off, sz)],
                                 dbuf.at[pl.ds(b * PC, sz)], isem[b])
                pltpu.async_copy(degp_hbm.at[pl.ds(ND_PAD + off, sz)],
                                 disbuf.at[pl.ds(b * PC, sz)], isem[b])

            def wait_pin(k, b):
                sz = psz(k)
                pltpu.make_async_copy(h_hbm.at[pl.ds(0, sz)],
                                      msgs[b].at[pl.ds(0, sz)],
                                      isem[b]).wait()
                pltpu.make_async_copy(degp_hbm.at[pl.ds(0, sz)],
                                      dbuf.at[pl.ds(b * PC, sz)],
                                      isem[b]).wait()
                pltpu.make_async_copy(degp_hbm.at[pl.ds(0, sz)],
                                      disbuf.at[pl.ds(b * PC, sz)],
                                      isem[b]).wait()

            def compute_p(k, b):
                sz = psz(k)

                def gbody(g, carry):
                    dv = (dbuf[pl.ds(b * PC + g * 16, 16)]
                          + disbuf[pl.ds(b * PC + g * 16, 16)] + 1.0)
                    y = _rsqrt16(dv)
                    for j in range(16):
                        e = g * 16 + j
                        msgs[b][PC + e, :] = jnp.zeros((HID,),
                                                       jnp.float32) + y[j]
                        msgs[b][e, :] = msgs[b][e, :] * y[j]
                    return carry

                lax.fori_loop(0, sz // 16, gbody, 0)

            def fire_pout(k, b):
                off = base + k * PC
                sz = psz(k)
                pltpu.async_copy(msgs[b].at[pl.ds(0, sz)],
                                 g_hbm.at[pl.ds(off, sz)], gsem[b])
                pltpu.async_copy(msgs[b].at[pl.ds(PC, sz)],
                                 de_hbm.at[pl.ds(off, sz)], gsem[b])

                @pl.when(c == 0)
                def _():
                    pltpu.sync_copy(msgs[b].at[pl.ds(0, sz)],
                                    acc.at[pl.ds(off, sz)])

            def drain_pout(k, b):
                sz = psz(k)
                pltpu.make_async_copy(g_hbm.at[pl.ds(0, sz)],
                                      msgs[b].at[pl.ds(0, sz)],
                                      gsem[b]).wait()
                pltpu.make_async_copy(g_hbm.at[pl.ds(0, sz)],
                                      msgs[b].at[pl.ds(PC, sz)],
                                      gsem[b]).wait()
        else:
            pltpu.sync_copy(b1_hbm, b1buf)

            def start_pin(k, b):
                off = base + k * PC
                sz = psz(k)
                pltpu.async_copy(pp0_hbm.at[pl.ds(off, sz)],
                                 msgs[b].at[pl.ds(0, sz)], isem[b])
                pltpu.async_copy(pp1_hbm.at[pl.ds(off, sz)],
                                 gbuf.at[pl.ds(b * PC, sz)], isem[b])
                pltpu.async_copy(de_hbm.at[pl.ds(off, sz)],
                                 msgs[b].at[pl.ds(PC, sz)], isem[b])

            def wait_pin(k, b):
                sz = psz(k)
                pltpu.make_async_copy(g_hbm.at[pl.ds(0, sz)],
                                      msgs[b].at[pl.ds(0, sz)],
                                      isem[b]).wait()
                pltpu.make_async_copy(g_hbm.at[pl.ds(0, sz)],
                                      gbuf.at[pl.ds(b * PC, sz)],
                                      isem[b]).wait()
                pltpu.make_async_copy(g_hbm.at[pl.ds(0, sz)],
                                      msgs[b].at[pl.ds(PC, sz)],
                                      isem[b]).wait()

            def compute_p(k, b):
                sz = psz(k)
                b1v = b1buf[...]

                def gbody(g, carry):
                    for j in range(16):
                        e = g * 16 + j
                        y = msgs[b][PC + e, :]
                        v = (msgs[b][e, :] + gbuf[b * PC + e, :]) * y + b1v
                        v = jnp.maximum(v, 0.0) * y
                        msgs[b][e, :] = v
                    return carry

                lax.fori_loop(0, sz // 16, gbody, 0)

            def fire_pout(k, b):
                off = base + k * PC
                sz = psz(k)
                pltpu.async_copy(msgs[b].at[pl.ds(0, sz)],
                                 g_hbm.at[pl.ds(off, sz)], gsem[b])

                @pl.when(c == 0)
                def _():
                    pltpu.sync_copy(msgs[b].at[pl.ds(0, sz)],
                                    acc.at[pl.ds(off, sz)])

            def drain_pout(k, b):
                sz = psz(k)
                pltpu.make_async_copy(g_hbm.at[pl.ds(0, sz)],
                                      msgs[b].at[pl.ds(0, sz)],
                                      gsem[b]).wait()

        start_pin(0, 0)
        for k in range(nchunks):
            b = k % 2
            wait_pin(k, b)
            if k >= 1:
                drain_pout(k - 1, 1 - b)
            if k + 1 < nchunks:
                start_pin(k + 1, 1 - b)
            compute_p(k, b)
            fire_pout(k, b)
        drain_pout(nchunks - 1, (nchunks - 1) % 2)

        @pl.when(c == 1)
        def _():
            def zbody(i, carry):
                msgs[0][i, :] = jnp.zeros((HID,), jnp.float32)
                return carry

            lax.fori_loop(0, CH, zbody, 0, unroll=8)
            for k in range(12):
                pltpu.async_copy(msgs[0], acc.at[pl.ds(base + k * CH, CH)],
                                 ssem[0])
            pltpu.async_copy(msgs[0].at[pl.ds(0, 112)],
                            acc.at[pl.ds(base + 12 * CH, 112)], ssem[0])
            for k in range(12):
                pltpu.make_async_copy(msgs[0],
                                      acc.at[pl.ds(base + k * CH, CH)],
                                      ssem[0]).wait()
            pltpu.make_async_copy(msgs[0].at[pl.ds(0, 112)],
                                  acc.at[pl.ds(base + 12 * CH, 112)],
                                  ssem[0]).wait()

        plsc.subcore_barrier()

        def start_in(rr, b, cs):
            ci = jnp.minimum(rr * NW + wid, NCH - 1)
            pltpu.async_copy(ei3_hbm.at[0, pl.ds(ci * CR, CR)], row_v[b],
                             isem[b])
            pltpu.async_copy(ew2d_hbm.at[pl.ds(ci * CR, CR)], ew_v[b],
                             isem[b])
            pltpu.async_copy(ei3_hbm.at[1, pl.ds(ci * CR, CR)], col_s[cs],
                             isem[b])

        def wait_in(b, cs):
            pltpu.make_async_copy(g_hbm.at[pl.ds(0, 96)],
                                  msgs[b].at[pl.ds(0, 96)],
                                  isem[b]).wait()

        def start_gather(b):
            for j in range(CR):
                pltpu.async_copy(g_hbm.at[row_v[b].at[j]],
                                 msgs[b].at[pl.ds(j * 128, 128)], gsem[b])

        start_in(0, 0, 0)
        start_in(1, 1, 1)
        wait_in(0, 0)
        start_gather(0)

        def six_body(r6, carry):
            for sub in range(6):
                b = sub % 2
                nb = 1 - b
                cs = sub % 3
                r = r6 * 6 + sub
                pltpu.make_async_copy(g_hbm.at[pl.ds(0, CH)], msgs[b],
                                      gsem[b]).wait()
                ci = r * NW + wid
                factor = jnp.where(ci < NCH, 1.0, 0.0)

                def scale_body(g, carry2):
                    jr = g // 8
                    g2 = g % 8
                    wv = ew_v[b][jr, pl.ds(g2 * 16, 16)] * factor
                    for j in range(16):
                        e = g * 16 + j
                        msgs[b][e, :] = msgs[b][e, :] * wv[j]
                    return carry2

                lax.fori_loop(0, CH // 16, scale_body, 0)
                for j in range(CR):
                    pltpu.async_copy(msgs[b].at[pl.ds(j * 128, 128)],
                                     acc.at[col_s[cs].at[j]], ssem[b],
                                     add=True)
                wait_in(nb, (cs + 1) % 3)
                if sub == 0:
                    @pl.when(r6 > 0)
                    def _():
                        pltpu.make_async_copy(g_hbm.at[pl.ds(0, CH)],
                                              msgs[nb], ssem[nb]).wait()
                else:
                    pltpu.make_async_copy(g_hbm.at[pl.ds(0, CH)], msgs[nb],
                                          ssem[nb]).wait()
                start_gather(nb)
                start_in(r + 2, b, (cs + 2) % 3)
            return carry

        lax.fori_loop(0, ROUNDS // 6, six_body, 0)
        pltpu.make_async_copy(g_hbm.at[pl.ds(0, CH)], msgs[1], ssem[1]).wait()
        pltpu.make_async_copy(g_hbm.at[pl.ds(0, CH)], msgs[0], gsem[0]).wait()
        wait_in(1, 1)
        plsc.subcore_barrier()

        myp = [p0_hbm, p1_hbm]
        for cc in range(NC):
            @pl.when(c == cc)
            def _(cc=cc):
                for k in range(13):
                    b = k % 2
                    sz = CH if k < 12 else 112
                    if k >= 2:
                        pltpu.make_async_copy(
                            msgs[b].at[pl.ds(0, CH)],
                            myp[cc].at[pl.ds(0, CH)], gsem[b]).wait()
                    pltpu.sync_copy(acc.at[pl.ds(base + k * CH, sz)],
                                    msgs[b].at[pl.ds(0, sz)])
                    pltpu.async_copy(msgs[b].at[pl.ds(0, sz)],
                                     myp[cc].at[pl.ds(base + k * CH, sz)],
                                     gsem[b])
                pltpu.make_async_copy(msgs[1].at[pl.ds(0, CH)],
                                      myp[cc].at[pl.ds(0, CH)],
                                      gsem[1]).wait()
                pltpu.make_async_copy(msgs[0].at[pl.ds(0, 112)],
                                      myp[cc].at[pl.ds(0, 112)],
                                      gsem[0]).wait()

    return pl.kernel(
        body,
        out_type=out_type,
        mesh=_mesh,
        compiler_params=_sc_params,
        scratch_types=scratch,
    )


_edge1 = _make_edge_kernel(True)
_edge2 = _make_edge_kernel(False)


BN = 2000


def _mm1_body(x_ref, w1_ref, o_ref):
    o_ref[...] = jnp.dot(x_ref[...], w1_ref[...],
                         preferred_element_type=jnp.float32)


def _mm1(x, W1):
    return pl.pallas_call(
        _mm1_body,
        grid=(N // BN,),
        in_specs=[
            pl.BlockSpec((BN, D_IN), lambda i: (i, 0)),
            pl.BlockSpec((D_IN, HID), lambda i: (0, 0)),
        ],
        out_specs=pl.BlockSpec((BN, HID), lambda i: (i, 0)),
        out_shape=jax.ShapeDtypeStruct((ND_PAD, HID), jnp.float32),
    )(x, W1)


FB = 250
FR = N * HID // 128
FRP = ND_PAD * HID // 128


def _fin_body(p0_ref, p1_ref, de_ref, w2_ref, b2_ref, o_ref):
    agg = (p0_ref[...] + p1_ref[...]) * de_ref[...]
    o_ref[...] = jnp.dot(agg, w2_ref[...],
                         preferred_element_type=jnp.float32) + b2_ref[...]


def _final(p0, p1, de, W2big, b2big):
    return pl.pallas_call(
        _fin_body,
        grid=(1,),
        in_specs=[
            pl.BlockSpec((FRP, 128), lambda i: (0, 0)),
            pl.BlockSpec((FRP, 128), lambda i: (0, 0)),
            pl.BlockSpec((FRP, 128), lambda i: (0, 0)),
            pl.BlockSpec((128, 8 * D_OUT), lambda i: (0, 0)),
            pl.BlockSpec((1, 8 * D_OUT), lambda i: (0, 0)),
        ],
        out_specs=pl.BlockSpec((FRP, 8 * D_OUT), lambda i: (0, 0)),
        out_shape=jax.ShapeDtypeStruct((FRP, 8 * D_OUT), jnp.float32),
    )(p0, p1, de, W2big, b2big)


def kernel(x, edge_index, edge_weight, W1, b1, W2, b2):
    ei3 = edge_index.reshape(2, E // 128, 128)
    ew2d = edge_weight.reshape(E // 128, 128)

    degp = _deg_kernel(ei3, ew2d)
    h = _mm1(x, W1)
    p10, p11, g1, de = _edge1(ei3, ew2d, degp, h)
    p20, p21, g2 = _edge2(ei3, ew2d, p10, p11, de, b1)
    W2big = jnp.kron(jnp.eye(8, dtype=jnp.float32), W2)
    b2big = jnp.tile(b2, 8).reshape(1, 8 * D_OUT)
    out = _final(p20.reshape(FRP, 128), p21.reshape(FRP, 128),
                 de.reshape(FRP, 128), W2big, b2big)
    return out.reshape(ND_PAD, D_OUT)[:N]


# speedup vs baseline: 62.4981x; 1.0020x over previous
import functools

import jax
import jax.numpy as jnp
from jax import lax
from jax.experimental import pallas as pl
from jax.experimental.pallas import tpu as pltpu
from jax.experimental.pallas import tpu_sc as plsc

N = 100000
E = 1600000
D_IN = 128
HID = 16
D_OUT = 40

NC = 2
NS = 16
NW = NC * NS
CH = 512
CR = CH // 128
NCH = E // CH
ROUNDS = 102
CHD = 2560
CRD = CHD // 128
NCHD = E // CHD
ROUNDS_D = 20
ND_PAD = 100096
NDPT = ND_PAD // NS
PC = 256
PNF = NDPT // PC
PREM = NDPT - PNF * PC

_mesh = plsc.VectorSubcoreMesh(core_axis_name="c", subcore_axis_name="s")
_sc_params = pltpu.CompilerParams(use_tc_tiling_on_sc=False,
                                  needs_layout_passes=False)


def _rsqrt16(d):
    i = plsc.bitcast(d, jnp.int32)
    y = plsc.bitcast(jnp.int32(0x5F3759DF) - (i >> 1), jnp.float32)
    for _ in range(3):
        y = y * (1.5 - 0.5 * d * y * y)
    return y


@functools.partial(
    pl.kernel,
    out_type=jax.ShapeDtypeStruct((NC * ND_PAD,), jnp.float32),
    mesh=_mesh,
    compiler_params=_sc_params,
    scratch_types=[
        pltpu.VMEM((CRD, 128), jnp.int32),
        pltpu.VMEM((CRD, 128), jnp.float32),
        pltpu.VMEM((NDPT,), jnp.float32),
        pltpu.VMEM_SHARED((ND_PAD,), jnp.float32),
        pltpu.SemaphoreType.DMA,
    ],
)
def _deg_kernel(ei3_hbm, ew2d_hbm, out_hbm, col_v, ew_v, zstage, acc, sem):
    c = lax.axis_index("c")
    s = lax.axis_index("s")
    wid = s * NC + c

    def zbody(i, carry):
        zstage[pl.ds(i * 16, 16)] = jnp.zeros((16,), jnp.float32)
        return carry

    lax.fori_loop(0, NDPT // 16, zbody, 0, unroll=8)
    base = s * NDPT
    pltpu.sync_copy(zstage, acc.at[pl.ds(base, NDPT)])
    plsc.subcore_barrier()

    def round_body(k, carry):
        ci = k * NW + wid

        @pl.when(ci < NCHD)
        def _():
            pltpu.sync_copy(ei3_hbm.at[1, pl.ds(ci * CRD, CRD)], col_v)
            pltpu.sync_copy(ew2d_hbm.at[pl.ds(ci * CRD, CRD)], ew_v)
            for j in range(CRD):
                pltpu.async_copy(ew_v.at[j], acc.at[col_v.at[j]], sem,
                                 add=True)
            pltpu.make_async_copy(ew2d_hbm.at[pl.ds(0, CRD)], ew_v,
                                  sem).wait()

        return carry

    lax.fori_loop(0, ROUNDS_D, round_body, 0)
    plsc.subcore_barrier()
    pltpu.sync_copy(acc.at[pl.ds(base, NDPT)], zstage)
    pltpu.sync_copy(zstage, out_hbm.at[pl.ds(c * ND_PAD + base, NDPT)])


def _make_edge_kernel(first):
    out_type = [
        jax.ShapeDtypeStruct((ND_PAD, HID), jnp.float32),
        jax.ShapeDtypeStruct((ND_PAD, HID), jnp.float32),
        jax.ShapeDtypeStruct((ND_PAD, HID), jnp.float32),
        jax.ShapeDtypeStruct((ND_PAD, HID), jnp.float32),
        jax.ShapeDtypeStruct((ND_PAD, HID), jnp.float32),
    ]
    if not first:
        out_type = out_type[:4]

    scratch = [
        pltpu.VMEM((CR, 128), jnp.int32), pltpu.VMEM((CR, 128), jnp.int32),
        pltpu.VMEM((CR, 128), jnp.int32), pltpu.VMEM((CR, 128), jnp.int32),
        pltpu.VMEM((CR, 128), jnp.int32),
        pltpu.VMEM((CR, 128), jnp.float32), pltpu.VMEM((CR, 128), jnp.float32),
        pltpu.VMEM((CH, HID), jnp.float32), pltpu.VMEM((CH, HID), jnp.float32),
        pltpu.VMEM((CH, HID), jnp.float32),
        pltpu.VMEM((CH,), jnp.float32),
        pltpu.VMEM((CH,), jnp.float32),
        pltpu.VMEM((16,), jnp.float32),
        pltpu.VMEM_SHARED((ND_PAD, HID), jnp.float32),
        pltpu.SemaphoreType.DMA, pltpu.SemaphoreType.DMA,
        pltpu.SemaphoreType.DMA, pltpu.SemaphoreType.DMA,
        pltpu.SemaphoreType.DMA, pltpu.SemaphoreType.DMA,
    ]

    def body(*refs):
        if first:
            (ei3_hbm, ew2d_hbm, degp_hbm, h_hbm,
             p0_hbm, p1_hbm, g0_hbm, g1_hbm, de_hbm,
             rv0, rv1, cv0, cv1, cv2, wv0, wv1, m0, m1, gbuf, dbuf, disbuf,
             b1buf, acc, is0, is1, gs0, gs1, ss0, ss1) = refs
        else:
            (ei3_hbm, ew2d_hbm, pp0_hbm, pp1_hbm, de_hbm, b1_hbm,
             p0_hbm, p1_hbm, g0_hbm, g1_hbm,
             rv0, rv1, cv0, cv1, cv2, wv0, wv1, m0, m1, gbuf, dbuf, disbuf,
             b1buf, acc, is0, is1, gs0, gs1, ss0, ss1) = refs

        row_v = [rv0, rv1]
        col_s = [cv0, cv1, cv2]
        ew_v = [wv0, wv1]
        msgs = [m0, m1]
        isem = [is0, is1]
        gsem = [gs0, gs1]
        ssem = [ss0, ss1]

        c = lax.axis_index("c")
        s = lax.axis_index("s")
        wid = s * NC + c
        base = s * NDPT
        g_hbm = g0_hbm
        gcs = [g0_hbm, g1_hbm]

        nchunks = PNF + 1

        def psz(k):
            return PC if k < PNF else PREM

        if first:
            def start_pin(k, b):
                off = base + k * PC
                sz = psz(k)
                pltpu.async_copy(h_hbm.at[pl.ds(off, sz)],
                                 msgs[b].at[pl.ds(0, sz)], isem[b])
                pltpu.async_copy(degp_hbm.at[pl.ds(off, sz)],
                                 dbuf.at[pl.ds(b * PC, sz)], isem[b])
                pltpu.async_copy(degp_hbm.at[pl.ds(ND_PAD + off, sz)],
                                 disbuf.at[pl.ds(b * PC, sz)], isem[b])

            def wait_pin(k, b):
                sz = psz(k)
                pltpu.make_async_copy(h_hbm.at[pl.ds(0, sz)],
                                      msgs[b].at[pl.ds(0, sz)],
                                      isem[b]).wait()
                pltpu.make_async_copy(degp_hbm.at[pl.ds(0, sz)],
                                      dbuf.at[pl.ds(b * PC, sz)],
                                      isem[b]).wait()
                pltpu.make_async_copy(degp_hbm.at[pl.ds(0, sz)],
                                      disbuf.at[pl.ds(b * PC, sz)],
                                      isem[b]).wait()

            def compute_p(k, b):
                sz = psz(k)

                def gbody(g, carry):
                    dv = (dbuf[pl.ds(b * PC + g * 16, 16)]
                          + disbuf[pl.ds(b * PC + g * 16, 16)] + 1.0)
                    y = _rsqrt16(dv)
                    for j in range(16):
                        e = g * 16 + j
                        msgs[b][PC + e, :] = jnp.zeros((HID,),
                                                       jnp.float32) + y[j]
                        msgs[b][e, :] = msgs[b][e, :] * y[j]
                    return carry

                lax.fori_loop(0, sz // 16, gbody, 0)

            def fire_pout(k, b):
                off = base + k * PC
                sz = psz(k)
                for cc in range(NC):
                    @pl.when(c == cc)
                    def _(cc=cc):
                        pltpu.async_copy(msgs[b].at[pl.ds(0, sz)],
                                         gcs[cc].at[pl.ds(off, sz)], gsem[b])
                pltpu.async_copy(msgs[b].at[pl.ds(PC, sz)],
                                 de_hbm.at[pl.ds(off, sz)], gsem[b])

                @pl.when(c == 0)
                def _():
                    pltpu.sync_copy(msgs[b].at[pl.ds(0, sz)],
                                    acc.at[pl.ds(off, sz)])

            def drain_pout(k, b):
                sz = psz(k)
                pltpu.make_async_copy(g_hbm.at[pl.ds(0, sz)],
                                      msgs[b].at[pl.ds(0, sz)],
                                      gsem[b]).wait()
                pltpu.make_async_copy(g_hbm.at[pl.ds(0, sz)],
                                      msgs[b].at[pl.ds(PC, sz)],
                                      gsem[b]).wait()
        else:
            pltpu.sync_copy(b1_hbm, b1buf)

            def start_pin(k, b):
                off = base + k * PC
                sz = psz(k)
                pltpu.async_copy(pp0_hbm.at[pl.ds(off, sz)],
                                 msgs[b].at[pl.ds(0, sz)], isem[b])
                pltpu.async_copy(pp1_hbm.at[pl.ds(off, sz)],
                                 gbuf.at[pl.ds(b * PC, sz)], isem[b])
                pltpu.async_copy(de_hbm.at[pl.ds(off, sz)],
                                 msgs[b].at[pl.ds(PC, sz)], isem[b])

            def wait_pin(k, b):
                sz = psz(k)
                pltpu.make_async_copy(g_hbm.at[pl.ds(0, sz)],
                                      msgs[b].at[pl.ds(0, sz)],
                                      isem[b]).wait()
                pltpu.make_async_copy(g_hbm.at[pl.ds(0, sz)],
                                      gbuf.at[pl.ds(b * PC, sz)],
                                      isem[b]).wait()
                pltpu.make_async_copy(g_hbm.at[pl.ds(0, sz)],
                                      msgs[b].at[pl.ds(PC, sz)],
                                      isem[b]).wait()

            def compute_p(k, b):
                sz = psz(k)
                b1v = b1buf[...]

                def gbody(g, carry):
                    for j in range(16):
                        e = g * 16 + j
                        y = msgs[b][PC + e, :]
                        v = (msgs[b][e, :] + gbuf[b * PC + e, :]) * y + b1v
                        v = jnp.maximum(v, 0.0) * y
                        msgs[b][e, :] = v
                    return carry

                lax.fori_loop(0, sz // 16, gbody, 0)

            def fire_pout(k, b):
                off = base + k * PC
                sz = psz(k)
                for cc in range(NC):
                    @pl.when(c == cc)
                    def _(cc=cc):
                        pltpu.async_copy(msgs[b].at[pl.ds(0, sz)],
                                         gcs[cc].at[pl.ds(off, sz)], gsem[b])

                @pl.when(c == 0)
                def _():
                    pltpu.sync_copy(msgs[b].at[pl.ds(0, sz)],
                                    acc.at[pl.ds(off, sz)])

            def drain_pout(k, b):
                sz = psz(k)
                pltpu.make_async_copy(g_hbm.at[pl.ds(0, sz)],
                                      msgs[b].at[pl.ds(0, sz)],
                                      gsem[b]).wait()

        start_pin(0, 0)
        for k in range(nchunks):
            b = k % 2
            wait_pin(k, b)
            if k >= 1:
                drain_pout(k - 1, 1 - b)
            if k + 1 < nchunks:
                start_pin(k + 1, 1 - b)
            compute_p(k, b)
            fire_pout(k, b)
        drain_pout(nchunks - 1, (nchunks - 1) % 2)

        @pl.when(c == 1)
        def _():
            def zbody(i, carry):
                msgs[0][i, :] = jnp.zeros((HID,), jnp.float32)
                return carry

            lax.fori_loop(0, CH, zbody, 0, unroll=8)
            for k in range(12):
                pltpu.async_copy(msgs[0], acc.at[pl.ds(base + k * CH, CH)],
                                 ssem[0])
            pltpu.async_copy(msgs[0].at[pl.ds(0, 112)],
                            acc.at[pl.ds(base + 12 * CH, 112)], ssem[0])
            for k in range(12):
                pltpu.make_async_copy(msgs[0],
                                      acc.at[pl.ds(base + k * CH, CH)],
                                      ssem[0]).wait()
            pltpu.make_async_copy(msgs[0].at[pl.ds(0, 112)],
                                  acc.at[pl.ds(base + 12 * CH, 112)],
                                  ssem[0]).wait()

        plsc.subcore_barrier()

        def start_in(rr, b, cs):
            ci = jnp.minimum(rr * NW + wid, NCH - 1)
            pltpu.async_copy(ei3_hbm.at[0, pl.ds(ci * CR, CR)], row_v[b],
                             isem[b])
            pltpu.async_copy(ew2d_hbm.at[pl.ds(ci * CR, CR)], ew_v[b],
                             isem[b])
            pltpu.async_copy(ei3_hbm.at[1, pl.ds(ci * CR, CR)], col_s[cs],
                             isem[b])

        def wait_in(b, cs):
            pltpu.make_async_copy(g_hbm.at[pl.ds(0, 96)],
                                  msgs[b].at[pl.ds(0, 96)],
                                  isem[b]).wait()

        def start_gather(b):
            for cc in range(NC):
                @pl.when(c == cc)
                def _(cc=cc, b=b):
                    for j in range(CR):
                        pltpu.async_copy(gcs[cc].at[row_v[b].at[j]],
                                         msgs[b].at[pl.ds(j * 128, 128)],
                                         gsem[b])

        start_in(0, 0, 0)
        start_in(1, 1, 1)
        wait_in(0, 0)
        start_gather(0)

        def six_body(r6, carry):
            for sub in range(6):
                b = sub % 2
                nb = 1 - b
                cs = sub % 3
                r = r6 * 6 + sub
                pltpu.make_async_copy(g_hbm.at[pl.ds(0, CH)], msgs[b],
                                      gsem[b]).wait()
                ci = r * NW + wid
                factor = jnp.where(ci < NCH, 1.0, 0.0)

                def scale_body(g, carry2):
                    jr = g // 8
                    g2 = g % 8
                    wv = ew_v[b][jr, pl.ds(g2 * 16, 16)] * factor
                    for j in range(16):
                        e = g * 16 + j
                        msgs[b][e, :] = msgs[b][e, :] * wv[j]
                    return carry2

                lax.fori_loop(0, CH // 16, scale_body, 0)
                for j in range(CR):
                    pltpu.async_copy(msgs[b].at[pl.ds(j * 128, 128)],
                                     acc.at[col_s[cs].at[j]], ssem[b],
                                     add=True)
                wait_in(nb, (cs + 1) % 3)
                if sub == 0:
                    @pl.when(r6 > 0)
                    def _():
                        pltpu.make_async_copy(g_hbm.at[pl.ds(0, CH)],
                                              msgs[nb], ssem[nb]).wait()
                else:
                    pltpu.make_async_copy(g_hbm.at[pl.ds(0, CH)], msgs[nb],
                                          ssem[nb]).wait()
                start_gather(nb)
                start_in(r + 2, b, (cs + 2) % 3)
            return carry

        lax.fori_loop(0, ROUNDS // 6, six_body, 0)
        pltpu.make_async_copy(g_hbm.at[pl.ds(0, CH)], msgs[1], ssem[1]).wait()
        pltpu.make_async_copy(g_hbm.at[pl.ds(0, CH)], msgs[0], gsem[0]).wait()
        wait_in(1, 1)
        plsc.subcore_barrier()

        myp = [p0_hbm, p1_hbm]
        for cc in range(NC):
            @pl.when(c == cc)
            def _(cc=cc):
                for k in range(13):
                    b = k % 2
                    sz = CH if k < 12 else 112
                    if k >= 2:
                        pltpu.make_async_copy(
                            msgs[b].at[pl.ds(0, CH)],
                            myp[cc].at[pl.ds(0, CH)], gsem[b]).wait()
                    pltpu.sync_copy(acc.at[pl.ds(base + k * CH, sz)],
                                    msgs[b].at[pl.ds(0, sz)])
                    pltpu.async_copy(msgs[b].at[pl.ds(0, sz)],
                                     myp[cc].at[pl.ds(base + k * CH, sz)],
                                     gsem[b])
                pltpu.make_async_copy(msgs[1].at[pl.ds(0, CH)],
                                      myp[cc].at[pl.ds(0, CH)],
                                      gsem[1]).wait()
                pltpu.make_async_copy(msgs[0].at[pl.ds(0, 112)],
                                      myp[cc].at[pl.ds(0, 112)],
                                      gsem[0]).wait()

    return pl.kernel(
        body,
        out_type=out_type,
        mesh=_mesh,
        compiler_params=_sc_params,
        scratch_types=scratch,
    )


_edge1 = _make_edge_kernel(True)
_edge2 = _make_edge_kernel(False)


BN = 2000


def _mm1_body(x_ref, w1_ref, o_ref):
    o_ref[...] = jnp.dot(x_ref[...], w1_ref[...],
                         preferred_element_type=jnp.float32)


def _mm1(x, W1):
    return pl.pallas_call(
        _mm1_body,
        grid=(N // BN,),
        in_specs=[
            pl.BlockSpec((BN, D_IN), lambda i: (i, 0)),
            pl.BlockSpec((D_IN, HID), lambda i: (0, 0)),
        ],
        out_specs=pl.BlockSpec((BN, HID), lambda i: (i, 0)),
        out_shape=jax.ShapeDtypeStruct((ND_PAD, HID), jnp.float32),
    )(x, W1)


FB = 250
FR = N * HID // 128
FRP = ND_PAD * HID // 128


def _fin_body(p0_ref, p1_ref, de_ref, w2_ref, b2_ref, o_ref):
    agg = (p0_ref[...] + p1_ref[...]) * de_ref[...]
    o_ref[...] = jnp.dot(agg, w2_ref[...],
                         preferred_element_type=jnp.float32) + b2_ref[...]


def _final(p0, p1, de, W2big, b2big):
    return pl.pallas_call(
        _fin_body,
        grid=(1,),
        in_specs=[
            pl.BlockSpec((FRP, 128), lambda i: (0, 0)),
            pl.BlockSpec((FRP, 128), lambda i: (0, 0)),
            pl.BlockSpec((FRP, 128), lambda i: (0, 0)),
            pl.BlockSpec((128, 8 * D_OUT), lambda i: (0, 0)),
            pl.BlockSpec((1, 8 * D_OUT), lambda i: (0, 0)),
        ],
        out_specs=pl.BlockSpec((FRP, 8 * D_OUT), lambda i: (0, 0)),
        out_shape=jax.ShapeDtypeStruct((FRP, 8 * D_OUT), jnp.float32),
    )(p0, p1, de, W2big, b2big)


def kernel(x, edge_index, edge_weight, W1, b1, W2, b2):
    ei3 = edge_index.reshape(2, E // 128, 128)
    ew2d = edge_weight.reshape(E // 128, 128)

    degp = _deg_kernel(ei3, ew2d)
    h = _mm1(x, W1)
    p10, p11, g1a, g1b, de = _edge1(ei3, ew2d, degp, h)
    p20, p21, g2a, g2b = _edge2(ei3, ew2d, p10, p11, de, b1)
    W2big = jnp.kron(jnp.eye(8, dtype=jnp.float32), W2)
    b2big = jnp.tile(b2, 8).reshape(1, 8 * D_OUT)
    out = _final(p20.reshape(FRP, 128), p21.reshape(FRP, 128),
                 de.reshape(FRP, 128), W2big, b2big)
    return out.reshape(ND_PAD, D_OUT)[:N]
